# Initial kernel scaffold; baseline (speedup 1.0000x reference)
import jax
import jax.numpy as jnp
from jax.experimental import pallas as pl


def kernel(x, edge_index, batch, W1, b1, W2, b2, Wmp, bmp, Wm1, bm1, Wm2, bm2, Ws, bs, V1, c1, V2, c2, Wl, bl):
    raise NotImplementedError("write your pallas kernel here")


# speedup vs baseline: 1.0086x; 1.0086x over previous
import functools

import jax
import jax.numpy as jnp
from jax.experimental import pallas as pl
from jax.experimental.pallas import tpu as pltpu

N_NODES = 100000
N_EDGES = 1600000
F_IN = 7
HID = 64
N_CLASSES = 2
N_GRAPHS = 20
RATIO = 0.5
BETA = 1.0
K_POOL = int(N_NODES * RATIO)
N_HOPS = 3

NB = 2000


def _mlp1_body(x_ref, agg_ref, W1_ref, b1_ref, W2_ref, b2_ref, h_ref):
    h0 = x_ref[...] + agg_ref[...]
    h1 = jnp.maximum(jnp.dot(h0, W1_ref[...], preferred_element_type=jnp.float32)
                     + b1_ref[...], 0.0)
    h_ref[...] = (jnp.dot(h1, W2_ref[...], preferred_element_type=jnp.float32)
                  + b2_ref[...])


def _gin_mlp(x, agg, W1, b1, W2, b2, n_rows):
    grid = (n_rows // NB,)
    f_in = x.shape[1]
    f_mid = W1.shape[1]
    f_out = W2.shape[1]
    return pl.pallas_call(
        _mlp1_body,
        grid=grid,
        in_specs=[
            pl.BlockSpec((NB, f_in), lambda i: (i, 0)),
            pl.BlockSpec((NB, f_in), lambda i: (i, 0)),
            pl.BlockSpec((f_in, f_mid), lambda i: (0, 0)),
            pl.BlockSpec((1, f_mid), lambda i: (0, 0)),
            pl.BlockSpec((f_mid, f_out), lambda i: (0, 0)),
            pl.BlockSpec((1, f_out), lambda i: (0, 0)),
        ],
        out_specs=pl.BlockSpec((NB, f_out), lambda i: (i, 0)),
        out_shape=jax.ShapeDtypeStruct((n_rows, f_out), jnp.float32),
    )(x, agg, W1, b1.reshape(1, -1), W2, b2.reshape(1, -1))


def _score_body(h_ref, aggh_ref, deg_ref, Wmp_ref, bmp_ref, Wm1_ref, bm1_ref,
                Wm2_ref, bm2_ref, Ws_ref, bs_ref, s_ref, hs_ref):
    h = h_ref[...]
    aggn = aggh_ref[...] / jnp.maximum(deg_ref[...], 1.0)
    z = jnp.maximum(jnp.dot(h + aggn, Wmp_ref[...], preferred_element_type=jnp.float32)
                    + bmp_ref[...], 0.0)
    z = jnp.maximum(jnp.dot(z, Wm1_ref[...], preferred_element_type=jnp.float32)
                    + bm1_ref[...], 0.0)
    z = jnp.maximum(jnp.dot(z, Wm2_ref[...], preferred_element_type=jnp.float32)
                    + bm2_ref[...], 0.0)
    s = jnp.tanh(jnp.dot(z, Ws_ref[...], preferred_element_type=jnp.float32)
                 + bs_ref[...])
    s_ref[...] = s
    hs_ref[...] = h * s


def _score_net(h, aggh, deg, Wmp, bmp, Wm1, bm1, Wm2, bm2, Ws, bs):
    grid = (N_NODES // NB,)
    s, hs = pl.pallas_call(
        _score_body,
        grid=grid,
        in_specs=[
            pl.BlockSpec((NB, HID), lambda i: (i, 0)),
            pl.BlockSpec((NB, HID), lambda i: (i, 0)),
            pl.BlockSpec((NB, 1), lambda i: (i, 0)),
            pl.BlockSpec((HID, HID), lambda i: (0, 0)),
            pl.BlockSpec((1, HID), lambda i: (0, 0)),
            pl.BlockSpec((HID, 32), lambda i: (0, 0)),
            pl.BlockSpec((1, 32), lambda i: (0, 0)),
            pl.BlockSpec((32, 32), lambda i: (0, 0)),
            pl.BlockSpec((1, 32), lambda i: (0, 0)),
            pl.BlockSpec((32, 1), lambda i: (0, 0)),
            pl.BlockSpec((1, 1), lambda i: (0, 0)),
        ],
        out_specs=[
            pl.BlockSpec((NB, 1), lambda i: (i, 0)),
            pl.BlockSpec((NB, HID), lambda i: (i, 0)),
        ],
        out_shape=[
            jax.ShapeDtypeStruct((N_NODES, 1), jnp.float32),
            jax.ShapeDtypeStruct((N_NODES, HID), jnp.float32),
        ],
    )(h, aggh, deg.reshape(-1, 1), Wmp, bmp.reshape(1, -1), Wm1,
      bm1.reshape(1, -1), Wm2, bm2.reshape(1, -1), Ws, bs.reshape(1, -1))
    return s[:, 0], hs


def kernel(x, edge_index, batch, W1, b1, W2, b2, Wmp, bmp, Wm1, bm1, Wm2, bm2,
           Ws, bs, V1, c1, V2, c2, Wl, bl):
    src, dst = edge_index[0], edge_index[1]
    n = x.shape[0]
    e = src.shape[0]

    agg1 = jax.ops.segment_sum(x[src], dst, num_segments=n)
    h = _gin_mlp(x, agg1, W1, b1, W2, b2, n)

    deg = jax.ops.segment_sum(jnp.ones((e,), jnp.float32), dst, num_segments=n)
    aggh = jax.ops.segment_sum(h[src], dst, num_segments=n)
    s, hs = _score_net(h, aggh, deg, Wmp, bmp, Wm1, bm1, Wm2, bm2, Ws, bs)

    mc_loss = BETA * jnp.sum(s[src] * s[dst]) / jnp.float32(e)

    _, idx = jax.lax.top_k(s, K_POOL)
    cluster = jnp.full((n,), -1, jnp.int32).at[idx].set(
        jnp.arange(K_POOL, dtype=jnp.int32))
    for _ in range(N_HOPS):
        keyv = jnp.where(cluster[src] >= 0, s[src], -jnp.inf)
        best = jax.ops.segment_max(keyv, dst, num_segments=n)
        cand = jnp.where((cluster[src] >= 0) & (keyv >= best[dst]),
                         cluster[src], -1)
        win = jax.ops.segment_max(cand, dst, num_segments=n)
        cluster = jnp.where(cluster >= 0, cluster, jnp.maximum(win, -1))
    cluster = jnp.where(cluster >= 0, cluster, 0)

    x_pool = jax.ops.segment_sum(hs, cluster, num_segments=K_POOL)
    batch_pool = batch[idx]

    new_src = cluster[src]
    new_dst = cluster[dst]
    mask = new_src != new_dst

    msg = jnp.where(mask[:, None], x_pool[new_src], 0.0)
    agg2 = jax.ops.segment_sum(msg, new_dst, num_segments=K_POOL)
    h2 = _gin_mlp(x_pool, agg2, V1, c1, V2, c2, K_POOL)

    cnt = jax.ops.segment_sum(jnp.ones((K_POOL,), jnp.float32), batch_pool,
                              num_segments=N_GRAPHS)
    pooled = jax.ops.segment_sum(h2, batch_pool, num_segments=N_GRAPHS) \
        / jnp.clip(cnt, 1.0)[:, None]
    logits = pooled @ Wl + bl
    return jax.nn.log_softmax(logits, axis=-1), mc_loss


# speedup vs baseline: 16.8523x; 16.7078x over previous
import functools

import jax
import jax.numpy as jnp
from jax import lax
from jax.experimental import pallas as pl
from jax.experimental.pallas import tpu as pltpu
from jax.experimental.pallas import tpu_sc as plsc

N_NODES = 100000
N_EDGES = 1600000
F_IN = 7
HID = 64
N_CLASSES = 2
N_GRAPHS = 20
RATIO = 0.5
BETA = 1.0
K_POOL = int(N_NODES * RATIO)
N_HOPS = 3

NC, NS, L = 2, 16, 16
NW = NC * NS
CW = 16

N_PAD = 100096
K_PAD = 50048
G_PAD = 32
NEG_INF = float("-inf")
I32_MIN = -2147483648
ZB = 208

_mesh = functools.partial(plsc.VectorSubcoreMesh,
                          core_axis_name="c", subcore_axis_name="s")
_SC_PARAMS = pltpu.CompilerParams(use_tc_tiling_on_sc=False, needs_layout_passes=False)


def _wid():
    return lax.axis_index("s") * NC + lax.axis_index("c")


def _fill_1d(ref, n, val, dtype):
    v = jnp.full((L,), val, dtype)

    def body(i, _):
        ref[pl.ds(i * L, L)] = v
        return 0

    lax.fori_loop(0, n // L, body, 0)


def _zero_stripe(acc, zbuf, row0, nrows, zb):
    full, rem = nrows // zb, nrows % zb
    for k in range(full):
        pltpu.sync_copy(zbuf, acc.at[pl.ds(row0 + k * zb, zb)])
    if rem:
        pltpu.sync_copy(zbuf.at[pl.ds(0, rem)],
                        acc.at[pl.ds(row0 + full * zb, rem)])


def _dump_stripe(acc, bounce, out_hbm, src_row0, dst_row0, nrows, bb):
    full, rem = nrows // bb, nrows % bb
    for k in range(full):
        pltpu.sync_copy(acc.at[pl.ds(src_row0 + k * bb, bb)], bounce)
        pltpu.sync_copy(bounce, out_hbm.at[pl.ds(dst_row0 + k * bb, bb)])
    if rem:
        pltpu.sync_copy(acc.at[pl.ds(src_row0 + full * bb, rem)],
                        bounce.at[pl.ds(0, rem)])
        pltpu.sync_copy(bounce.at[pl.ds(0, rem)],
                        out_hbm.at[pl.ds(dst_row0 + full * bb, rem)])


def _seg_sum_partial(table, src, dst, n_out_pad, eb):
    e = src.shape[0]
    e_pw = e // NW
    nb = e_pw // eb
    stripe = n_out_pad // NS

    @functools.partial(
        pl.kernel,
        out_type=jax.ShapeDtypeStruct((NC * n_out_pad, CW), jnp.float32),
        mesh=_mesh(),
        compiler_params=_SC_PARAMS,
        scratch_types=[
            pltpu.VMEM_SHARED((n_out_pad, CW), jnp.float32),
            pltpu.VMEM((eb,), jnp.int32),
            pltpu.VMEM((eb,), jnp.int32),
            pltpu.VMEM((eb, CW), jnp.float32),
            pltpu.SemaphoreType.DMA,
        ],
    )
    def k(table_h, src_h, dst_h, out_h, acc, idxs, idxd, rows, sem):
        cid = lax.axis_index("c")
        sid = lax.axis_index("s")
        wid = sid * NC + cid
        def zb(i, _):
            rows[i, :] = jnp.zeros((L,), jnp.float32)
            return 0
        lax.fori_loop(0, eb, zb, 0)
        _zero_stripe(acc, rows, sid * stripe, stripe, eb)
        plsc.subcore_barrier()

        base = wid * e_pw

        def body(b, _):
            off = base + b * eb
            pltpu.sync_copy(src_h.at[pl.ds(off, eb)], idxs)
            pltpu.sync_copy(dst_h.at[pl.ds(off, eb)], idxd)
            pltpu.async_copy(table_h.at[idxs], rows, sem).wait()
            pltpu.sync_copy(rows, acc.at[idxd], add=True)
            return 0

        lax.fori_loop(0, nb, body, 0)
        plsc.subcore_barrier()
        _dump_stripe(acc, rows, out_h, sid * stripe,
                     cid * n_out_pad + sid * stripe, stripe, eb)

    return k(table, src, dst)


def _seg_sum_chunks(tables, src, dst, n_out_pad, eb):
    n_chunks = len(tables)
    e = src.shape[0]
    e_pt = e // NS
    nb = e_pt // eb
    stripe = n_out_pad // NS

    @functools.partial(
        pl.kernel,
        out_type=jax.ShapeDtypeStruct((n_chunks * n_out_pad, CW), jnp.float32),
        mesh=_mesh(),
        compiler_params=_SC_PARAMS,
        scratch_types=[
            pltpu.VMEM_SHARED((n_out_pad, CW), jnp.float32),
            pltpu.VMEM((eb,), jnp.int32),
            pltpu.VMEM((eb,), jnp.int32),
            pltpu.VMEM((eb, CW), jnp.float32),
            pltpu.VMEM((ZB, CW), jnp.float32),
            pltpu.SemaphoreType.DMA,
        ],
    )
    def k(*refs):
        tabs = refs[:n_chunks]
        src_h, dst_h, out_h, acc, idxs, idxd, rows, zbuf, sem = refs[n_chunks:]
        cid = lax.axis_index("c")
        sid = lax.axis_index("s")

        def zb(i, _):
            zbuf[i, :] = jnp.zeros((L,), jnp.float32)
            return 0
        lax.fori_loop(0, ZB, zb, 0)

        base = sid * e_pt
        for c in range(n_chunks):
            on = (cid == (c % NC))

            @pl.when(on)
            def _():
                _zero_stripe(acc, zbuf, sid * stripe, stripe, ZB)
            plsc.subcore_barrier()

            @pl.when(on)
            def _():
                def body(b, _):
                    off = base + b * eb
                    pltpu.sync_copy(src_h.at[pl.ds(off, eb)], idxs)
                    pltpu.sync_copy(dst_h.at[pl.ds(off, eb)], idxd)
                    pltpu.async_copy(tabs[c].at[idxs], rows, sem).wait()
                    pltpu.sync_copy(rows, acc.at[idxd], add=True)
                    return 0
                lax.fori_loop(0, nb, body, 0)
            plsc.subcore_barrier()

            @pl.when(on)
            def _():
                _dump_stripe(acc, rows, out_h, sid * stripe,
                             c * n_out_pad + sid * stripe, stripe, eb)
            plsc.subcore_barrier()

    return k(*tables, src, dst)


def _translate_edges(cluster_pad, src, dst, trash, eb):
    n = cluster_pad.shape[0]
    e = src.shape[0]
    e_pw = e // NW
    nb = e_pw // eb

    @functools.partial(
        pl.kernel,
        out_type=(jax.ShapeDtypeStruct((e,), jnp.int32),
                  jax.ShapeDtypeStruct((e,), jnp.int32)),
        mesh=_mesh(),
        compiler_params=_SC_PARAMS,
        scratch_types=[
            pltpu.VMEM((n,), jnp.int32),
            pltpu.VMEM((eb,), jnp.int32),
            pltpu.VMEM((eb,), jnp.int32),
            pltpu.VMEM((eb,), jnp.int32),
            pltpu.VMEM((eb,), jnp.int32),
        ],
    )
    def k(cl_h, src_h, dst_h, nsrc_h, mdst_h, cl, idxs, idxd, obs, obd):
        wid = _wid()
        pltpu.sync_copy(cl_h, cl)
        base = wid * e_pw
        trash_v = jnp.full((L,), trash, jnp.int32)

        def body(b, _):
            off = base + b * eb
            pltpu.sync_copy(src_h.at[pl.ds(off, eb)], idxs)
            pltpu.sync_copy(dst_h.at[pl.ds(off, eb)], idxd)

            def inner(j, _):
                s16 = idxs[pl.ds(j * L, L)]
                d16 = idxd[pl.ds(j * L, L)]
                cs = plsc.load_gather(cl, [s16])
                cd = plsc.load_gather(cl, [d16])
                obs[pl.ds(j * L, L)] = cs
                obd[pl.ds(j * L, L)] = jnp.where(cs == cd, trash_v, cd)
                return 0

            lax.fori_loop(0, eb // L, inner, 0)
            pltpu.sync_copy(obs, nsrc_h.at[pl.ds(off, eb)])
            pltpu.sync_copy(obd, mdst_h.at[pl.ds(off, eb)])
            return 0

        lax.fori_loop(0, nb, body, 0)

    return k(cluster_pad, src, dst)


def _edge_dot(s_pad, src, dst, eb):
    n = s_pad.shape[0]
    e = src.shape[0]
    e_pw = e // NW
    nb = e_pw // eb

    @functools.partial(
        pl.kernel,
        out_type=jax.ShapeDtypeStruct((NW * L,), jnp.float32),
        mesh=_mesh(),
        compiler_params=_SC_PARAMS,
        scratch_types=[
            pltpu.VMEM((n,), jnp.float32),
            pltpu.VMEM((eb,), jnp.int32),
            pltpu.VMEM((eb,), jnp.int32),
            pltpu.VMEM((L,), jnp.float32),
        ],
    )
    def k(s_h, src_h, dst_h, out_h, sv, idxs, idxd, accb):
        wid = _wid()
        pltpu.sync_copy(s_h, sv)
        base = wid * e_pw

        def body(b, acc):
            off = base + b * eb
            pltpu.sync_copy(src_h.at[pl.ds(off, eb)], idxs)
            pltpu.sync_copy(dst_h.at[pl.ds(off, eb)], idxd)

            def inner(j, a):
                s16 = idxs[pl.ds(j * L, L)]
                d16 = idxd[pl.ds(j * L, L)]
                return a + (plsc.load_gather(sv, [s16])
                            * plsc.load_gather(sv, [d16]))

            return lax.fori_loop(0, eb // L, inner, acc)

        acc = lax.fori_loop(0, nb, body, jnp.zeros((L,), jnp.float32))
        accb[...] = acc
        pltpu.sync_copy(accb, out_h.at[pl.ds(wid * L, L)])

    return k(s_pad, src, dst)


HALF = N_PAD // 2


def _rmw_max(acc, d16, v16, active):
    def cond(p):
        return jnp.any(p)

    def body(p):
        cur = plsc.load_gather(acc, [d16], mask=p)
        upd = p & (v16 > cur)
        plsc.store_scatter(acc, [d16], v16, mask=upd)
        chk = plsc.load_gather(acc, [d16], mask=upd)
        return upd & (chk < v16)

    lax.while_loop(cond, body, active)


def _hop_best(skey_pad, src, dst, h0, eb):
    e = src.shape[0]
    e_pw = e // NW
    nb = e_pw // eb

    @functools.partial(
        pl.kernel,
        out_type=jax.ShapeDtypeStruct((NW * HALF,), jnp.float32),
        mesh=_mesh(),
        compiler_params=_SC_PARAMS,
        scratch_types=[
            pltpu.VMEM((HALF,), jnp.float32),
            pltpu.VMEM((eb,), jnp.int32),
            pltpu.VMEM((eb,), jnp.int32),
            pltpu.VMEM((eb,), jnp.float32),
            pltpu.SemaphoreType.DMA,
        ],
    )
    def k(skey_h, src_h, dst_h, out_h, acc, idxs, idxd, kv, sem):
        wid = _wid()
        _fill_1d(acc, HALF, NEG_INF, jnp.float32)
        base = wid * e_pw
        lo = jnp.full((L,), h0, jnp.int32)
        hi_m1 = jnp.full((L,), HALF - 1, jnp.int32)
        zero = jnp.zeros((L,), jnp.int32)

        def body(b, _):
            off = base + b * eb
            pltpu.sync_copy(src_h.at[pl.ds(off, eb)], idxs)
            pltpu.async_copy(skey_h.at[idxs], kv, sem).wait()
            pltpu.sync_copy(dst_h.at[pl.ds(off, eb)], idxd)

            def inner(j, _):
                d16 = idxd[pl.ds(j * L, L)] - lo
                m = (d16 >= zero) & (d16 <= hi_m1)
                dl = jnp.clip(d16, 0, HALF - 1)
                v16 = kv[pl.ds(j * L, L)]
                _rmw_max(acc, dl, v16, m)
                return 0

            lax.fori_loop(0, eb // L, inner, 0)
            return 0

        lax.fori_loop(0, nb, body, 0)
        pltpu.sync_copy(acc, out_h.at[pl.ds(wid * HALF, HALF)])

    return k(skey_pad, src, dst)


def _hop_win(skey_pad, cluster_pad, best_pad, src, dst, h0, eb):
    e = src.shape[0]
    e_pw = e // NW
    nb = e_pw // eb

    @functools.partial(
        pl.kernel,
        out_type=jax.ShapeDtypeStruct((NW * HALF,), jnp.int32),
        mesh=_mesh(),
        compiler_params=_SC_PARAMS,
        scratch_types=[
            pltpu.VMEM((HALF,), jnp.int32),
            pltpu.VMEM((HALF,), jnp.float32),
            pltpu.VMEM((eb,), jnp.int32),
            pltpu.VMEM((eb,), jnp.int32),
            pltpu.VMEM((eb,), jnp.float32),
            pltpu.VMEM((eb,), jnp.int32),
            pltpu.SemaphoreType.DMA,
        ],
    )
    def k(skey_h, cl_h, best_h, src_h, dst_h, out_h,
          acc, bestv, idxs, idxd, kv, cv, sem):
        wid = _wid()
        _fill_1d(acc, HALF, I32_MIN, jnp.int32)
        pltpu.sync_copy(best_h.at[pl.ds(h0, HALF)], bestv)
        base = wid * e_pw
        lo = jnp.full((L,), h0, jnp.int32)
        hi_m1 = jnp.full((L,), HALF - 1, jnp.int32)
        zero = jnp.zeros((L,), jnp.int32)
        min_v = jnp.full((L,), I32_MIN, jnp.int32)

        def body(b, _):
            off = base + b * eb
            pltpu.sync_copy(src_h.at[pl.ds(off, eb)], idxs)
            pltpu.async_copy(skey_h.at[idxs], kv, sem).wait()
            pltpu.async_copy(cl_h.at[idxs], cv, sem).wait()
            pltpu.sync_copy(dst_h.at[pl.ds(off, eb)], idxd)

            def inner(j, _):
                d16 = idxd[pl.ds(j * L, L)] - lo
                m = (d16 >= zero) & (d16 <= hi_m1)
                dl = jnp.clip(d16, 0, HALF - 1)
                k16 = kv[pl.ds(j * L, L)]
                c16 = cv[pl.ds(j * L, L)]
                b16 = plsc.load_gather(bestv, [dl], mask=m)
                v16 = jnp.where(k16 >= b16, c16, min_v)
                _rmw_max(acc, dl, v16, m)
                return 0

            lax.fori_loop(0, eb // L, inner, 0)
            return 0

        lax.fori_loop(0, nb, body, 0)
        pltpu.sync_copy(acc, out_h.at[pl.ds(wid * HALF, HALF)])

    return k(skey_pad, cluster_pad, best_pad, src, dst)


NB = 2000


def _gin_body(x_ref, a0_ref, a1_ref, W1_ref, b1_ref, W2_ref, b2_ref, *out_refs):
    h0 = x_ref[...] + a0_ref[...] + a1_ref[...]
    h1 = jnp.maximum(jnp.dot(h0, W1_ref[...], preferred_element_type=jnp.float32)
                     + b1_ref[...], 0.0)
    h = jnp.dot(h1, W2_ref[...], preferred_element_type=jnp.float32) + b2_ref[...]
    for c, r in enumerate(out_refs):
        r[...] = h[:, c * CW:(c + 1) * CW]


def _gin_mlp_chunked(x, a0, a1, W1, b1, W2, b2, n_rows):
    grid = (n_rows // NB,)
    f_in = x.shape[1]
    f_mid = W1.shape[1]
    n_out_chunks = HID // CW
    return pl.pallas_call(
        _gin_body,
        grid=grid,
        in_specs=[
            pl.BlockSpec((NB, f_in), lambda i: (i, 0)),
            pl.BlockSpec((NB, f_in), lambda i: (i, 0)),
            pl.BlockSpec((NB, f_in), lambda i: (i, 0)),
            pl.BlockSpec((f_in, f_mid), lambda i: (0, 0)),
            pl.BlockSpec((1, f_mid), lambda i: (0, 0)),
            pl.BlockSpec((f_mid, HID), lambda i: (0, 0)),
            pl.BlockSpec((1, HID), lambda i: (0, 0)),
        ],
        out_specs=[pl.BlockSpec((NB, CW), lambda i: (i, 0))
                   for _ in range(n_out_chunks)],
        out_shape=[jax.ShapeDtypeStruct((n_rows, CW), jnp.float32)
                   for _ in range(n_out_chunks)],
    )(x, a0, a1, W1, b1.reshape(1, -1), W2, b2.reshape(1, -1))


def _score_body(h0_ref, h1_ref, h2_ref, h3_ref, g0_ref, g1_ref, g2_ref, g3_ref,
                deg_ref, Wmp_ref, bmp_ref, Wm1_ref, bm1_ref, Wm2_ref, bm2_ref,
                Ws_ref, bs_ref, s_ref, *hs_refs):
    h = jnp.concatenate([h0_ref[...], h1_ref[...], h2_ref[...], h3_ref[...]],
                        axis=1)
    agg = jnp.concatenate([g0_ref[...], g1_ref[...], g2_ref[...], g3_ref[...]],
                          axis=1)
    aggn = agg / jnp.maximum(deg_ref[...], 1.0)
    z = jnp.maximum(jnp.dot(h + aggn, Wmp_ref[...],
                            preferred_element_type=jnp.float32) + bmp_ref[...], 0.0)
    z = jnp.maximum(jnp.dot(z, Wm1_ref[...],
                            preferred_element_type=jnp.float32) + bm1_ref[...], 0.0)
    z = jnp.maximum(jnp.dot(z, Wm2_ref[...],
                            preferred_element_type=jnp.float32) + bm2_ref[...], 0.0)
    s = jnp.tanh(jnp.dot(z, Ws_ref[...],
                         preferred_element_type=jnp.float32) + bs_ref[...])
    s_ref[...] = s
    hs = h * s
    for c, r in enumerate(hs_refs):
        r[...] = hs[:, c * CW:(c + 1) * CW]


def _score_net(hc, aggc, deg, Wmp, bmp, Wm1, bm1, Wm2, bm2, Ws, bs):
    grid = (N_NODES // NB,)
    outs = pl.pallas_call(
        _score_body,
        grid=grid,
        in_specs=(
            [pl.BlockSpec((NB, CW), lambda i: (i, 0)) for _ in range(4)]
            + [pl.BlockSpec((NB, CW), lambda i: (i, 0)) for _ in range(4)]
            + [pl.BlockSpec((NB, 1), lambda i: (i, 0)),
               pl.BlockSpec((HID, HID), lambda i: (0, 0)),
               pl.BlockSpec((1, HID), lambda i: (0, 0)),
               pl.BlockSpec((HID, 32), lambda i: (0, 0)),
               pl.BlockSpec((1, 32), lambda i: (0, 0)),
               pl.BlockSpec((32, 32), lambda i: (0, 0)),
               pl.BlockSpec((1, 32), lambda i: (0, 0)),
               pl.BlockSpec((32, 1), lambda i: (0, 0)),
               pl.BlockSpec((1, 1), lambda i: (0, 0))]
        ),
        out_specs=[pl.BlockSpec((NB, 1), lambda i: (i, 0))]
        + [pl.BlockSpec((NB, CW), lambda i: (i, 0)) for _ in range(4)],
        out_shape=[jax.ShapeDtypeStruct((N_NODES, 1), jnp.float32)]
        + [jax.ShapeDtypeStruct((N_NODES, CW), jnp.float32) for _ in range(4)],
    )(*hc, *aggc, deg.reshape(-1, 1), Wmp, bmp.reshape(1, -1), Wm1,
      bm1.reshape(1, -1), Wm2, bm2.reshape(1, -1), Ws, bs.reshape(1, -1))
    return outs[0][:, 0], outs[1:]


def _logits_body(p0, p1, p2, p3, cnt_ref, Wl_ref, bl_ref, out_ref):
    pooled = jnp.concatenate([p0[...], p1[...], p2[...], p3[...]], axis=1)
    pooled = pooled / jnp.maximum(cnt_ref[...], 1.0)
    logits = jnp.dot(pooled, Wl_ref[...],
                     preferred_element_type=jnp.float32) + bl_ref[...]
    m = jnp.max(logits, axis=1, keepdims=True)
    sh = logits - m
    lse = jnp.log(jnp.sum(jnp.exp(sh), axis=1, keepdims=True))
    out_ref[...] = sh - lse


def _logits(pooled_chunks, cnt, Wl, bl):
    return pl.pallas_call(
        _logits_body,
        grid=(1,),
        in_specs=[pl.BlockSpec((G_PAD, CW), lambda i: (0, 0)) for _ in range(4)]
        + [pl.BlockSpec((G_PAD, 1), lambda i: (0, 0)),
           pl.BlockSpec((HID, N_CLASSES), lambda i: (0, 0)),
           pl.BlockSpec((1, N_CLASSES), lambda i: (0, 0))],
        out_specs=pl.BlockSpec((G_PAD, N_CLASSES), lambda i: (0, 0)),
        out_shape=jax.ShapeDtypeStruct((G_PAD, N_CLASSES), jnp.float32),
    )(*pooled_chunks, cnt, Wl, bl.reshape(1, -1))


def _pad_rows(a, n_pad):
    return jnp.pad(a, ((0, n_pad - a.shape[0]), (0, 0)))


def _pad_1d(a, n_pad, val=0):
    return jnp.pad(a, (0, n_pad - a.shape[0]), constant_values=val)


def kernel(x, edge_index, batch, W1, b1, W2, b2, Wmp, bmp, Wm1, bm1, Wm2, bm2,
           Ws, bs, V1, c1, V2, c2, Wl, bl):
    src, dst = edge_index[0], edge_index[1]
    n = x.shape[0]
    e = src.shape[0]

    x16 = jnp.concatenate(
        [x, jnp.ones((n, 1), jnp.float32), jnp.zeros((n, CW - F_IN - 1), jnp.float32)],
        axis=1)
    x16p = _pad_rows(x16, N_PAD)
    agg1p = _seg_sum_partial(x16p, src, dst, N_PAD, 1000)
    agg1p = agg1p.reshape(NC, N_PAD, CW)
    a0 = agg1p[0, :n, :]
    a1 = agg1p[1, :n, :]
    deg = a0[:, F_IN] + a1[:, F_IN]
    W1p = jnp.pad(W1, ((0, CW - F_IN), (0, 0)))
    hc = _gin_mlp_chunked(x16, a0, a1, W1p, b1, W2, b2, n)

    hcp = [_pad_rows(h, N_PAD) for h in hc]
    aggh = _seg_sum_chunks(hcp, src, dst, N_PAD, 1000).reshape(4, N_PAD, CW)
    aggc = [aggh[c, :n, :] for c in range(4)]
    s, hs_chunks = _score_net(hc, aggc, deg, Wmp, bmp, Wm1, bm1, Wm2, bm2, Ws, bs)

    sp = _pad_1d(s, N_PAD)
    mc_parts = _edge_dot(sp, src, dst, 2000)
    mc_loss = BETA * jnp.sum(mc_parts) / jnp.float32(e)

    _, idx = jax.lax.top_k(s, K_POOL)
    cluster = jnp.full((n,), -1, jnp.int32).at[idx].set(
        jnp.arange(K_POOL, dtype=jnp.int32))

    for _ in range(N_HOPS):
        skey = jnp.where(cluster >= 0, s, NEG_INF)
        skp = _pad_1d(skey, N_PAD, NEG_INF)
        clp = _pad_1d(cluster, N_PAD, -1)
        halves = []
        for h in range(2):
            bp = _hop_best(skp, src, dst, h * HALF, 2000).reshape(NW, HALF)
            halves.append(jnp.max(bp, axis=0))
        best = jnp.concatenate(halves)
        wins = []
        for h in range(2):
            wp = _hop_win(skp, clp, best, src, dst, h * HALF, 2000).reshape(NW, HALF)
            wins.append(jnp.max(wp, axis=0))
        win = jnp.concatenate(wins)[:n]
        cluster = jnp.where(cluster >= 0, cluster, jnp.maximum(win, -1))
    cluster = jnp.where(cluster >= 0, cluster, 0)

    E_XP = N_PAD
    hs_pad = [_pad_rows(c, N_PAD) for c in hs_chunks]
    lin_idx = jnp.arange(E_XP, dtype=jnp.int32)
    xp_dst = _pad_1d(cluster, E_XP, K_POOL)
    x_pool = _seg_sum_chunks(hs_pad, lin_idx, xp_dst, K_PAD, 3128)
    x_pool = x_pool.reshape(4, K_PAD, CW)
    batch_pool = batch[idx]

    clp = _pad_1d(cluster, N_PAD, 0)
    nsrc, mdst = _translate_edges(clp, src, dst, K_POOL, 2000)
    xp_chunks = [x_pool[c] for c in range(4)]
    agg2 = _seg_sum_chunks(xp_chunks, nsrc, mdst, K_PAD, 1000)
    agg2 = agg2.reshape(4, K_PAD, CW)
    xpc = jnp.concatenate([x_pool[c, :K_POOL, :] for c in range(4)], axis=1)
    a2 = jnp.concatenate([agg2[c, :K_POOL, :] for c in range(4)], axis=1)
    zeros_k = jnp.zeros((K_POOL, HID), jnp.float32)
    h2c = _gin_mlp_chunked(xpc, a2, zeros_k, V1, c1, V2, c2, K_POOL)

    h2p = [_pad_rows(c, K_PAD) for c in h2c]
    ones_chunk = jnp.zeros((K_PAD, CW), jnp.float32).at[:K_POOL, 0].set(1.0)
    lin_k = jnp.arange(K_PAD, dtype=jnp.int32)
    ro_dst = _pad_1d(batch_pool, K_PAD, N_GRAPHS)
    ro = _seg_sum_chunks(h2p + [ones_chunk], lin_k, ro_dst, G_PAD, 3128)
    ro = ro.reshape(5, G_PAD, CW)
    pooled_chunks = [ro[c] for c in range(4)]
    cnt = ro[4, :, 0:1]
    out = _logits(pooled_chunks, cnt, Wl, bl)
    return out[:N_GRAPHS], mc_loss


# speedup vs baseline: 22.7457x; 1.3497x over previous
import functools

import jax
import jax.numpy as jnp
from jax import lax
from jax.experimental import pallas as pl
from jax.experimental.pallas import tpu as pltpu
from jax.experimental.pallas import tpu_sc as plsc

N_NODES = 100000
N_EDGES = 1600000
F_IN = 7
HID = 64
N_CLASSES = 2
N_GRAPHS = 20
RATIO = 0.5
BETA = 1.0
K_POOL = int(N_NODES * RATIO)
N_HOPS = 3

NC, NS, L = 2, 16, 16
NW = NC * NS
CW = 16

N_PAD = 100096
K_PAD = 50048
G_PAD = 32
NEG_INF = float("-inf")
I32_MIN = -2147483648
ZB = 208

_mesh = functools.partial(plsc.VectorSubcoreMesh,
                          core_axis_name="c", subcore_axis_name="s")
_SC_PARAMS = pltpu.CompilerParams(use_tc_tiling_on_sc=False, needs_layout_passes=False)


def _wid():
    return lax.axis_index("s") * NC + lax.axis_index("c")


def _fill_1d(ref, n, val, dtype):
    v = jnp.full((L,), val, dtype)

    def body(i, _):
        ref[pl.ds(i * L, L)] = v
        return 0

    lax.fori_loop(0, n // L, body, 0)


def _zero_stripe(acc, zbuf, row0, nrows, zb):
    full, rem = nrows // zb, nrows % zb
    for k in range(full):
        pltpu.sync_copy(zbuf, acc.at[pl.ds(row0 + k * zb, zb)])
    if rem:
        pltpu.sync_copy(zbuf.at[pl.ds(0, rem)],
                        acc.at[pl.ds(row0 + full * zb, rem)])


def _dump_stripe(acc, bounce, out_hbm, src_row0, dst_row0, nrows, bb):
    full, rem = nrows // bb, nrows % bb
    for k in range(full):
        pltpu.sync_copy(acc.at[pl.ds(src_row0 + k * bb, bb)], bounce)
        pltpu.sync_copy(bounce, out_hbm.at[pl.ds(dst_row0 + k * bb, bb)])
    if rem:
        pltpu.sync_copy(acc.at[pl.ds(src_row0 + full * bb, rem)],
                        bounce.at[pl.ds(0, rem)])
        pltpu.sync_copy(bounce.at[pl.ds(0, rem)],
                        out_hbm.at[pl.ds(dst_row0 + full * bb, rem)])


def _seg_sum_partial(table, src, dst, n_out_pad, eb):
    e = src.shape[0]
    e_pw = e // NW
    nb = e_pw // eb
    stripe = n_out_pad // NS

    @functools.partial(
        pl.kernel,
        out_type=jax.ShapeDtypeStruct((NC * n_out_pad, CW), jnp.float32),
        mesh=_mesh(),
        compiler_params=_SC_PARAMS,
        scratch_types=[
            pltpu.VMEM_SHARED((n_out_pad, CW), jnp.float32),
            pltpu.VMEM((eb,), jnp.int32),
            pltpu.VMEM((eb,), jnp.int32),
            pltpu.VMEM((eb, CW), jnp.float32),
            pltpu.SemaphoreType.DMA,
        ],
    )
    def k(table_h, src_h, dst_h, out_h, acc, idxs, idxd, rows, sem):
        cid = lax.axis_index("c")
        sid = lax.axis_index("s")
        wid = sid * NC + cid
        def zb(i, _):
            rows[i, :] = jnp.zeros((L,), jnp.float32)
            return 0
        lax.fori_loop(0, eb, zb, 0)
        _zero_stripe(acc, rows, sid * stripe, stripe, eb)
        plsc.subcore_barrier()

        base = wid * e_pw

        def body(b, _):
            off = base + b * eb
            pltpu.sync_copy(src_h.at[pl.ds(off, eb)], idxs)
            pltpu.sync_copy(dst_h.at[pl.ds(off, eb)], idxd)
            pltpu.async_copy(table_h.at[idxs], rows, sem).wait()
            pltpu.sync_copy(rows, acc.at[idxd], add=True)
            return 0

        lax.fori_loop(0, nb, body, 0)
        plsc.subcore_barrier()
        _dump_stripe(acc, rows, out_h, sid * stripe,
                     cid * n_out_pad + sid * stripe, stripe, eb)

    return k(table, src, dst)


def _seg_sum_chunks(tables, src, dst, n_out_pad, eb):
    n_chunks = len(tables)
    e = src.shape[0]
    e_pt = e // NS
    nb = e_pt // eb
    stripe = n_out_pad // NS

    @functools.partial(
        pl.kernel,
        out_type=jax.ShapeDtypeStruct((n_chunks * n_out_pad, CW), jnp.float32),
        mesh=_mesh(),
        compiler_params=_SC_PARAMS,
        scratch_types=[
            pltpu.VMEM_SHARED((n_out_pad, CW), jnp.float32),
            pltpu.VMEM((eb,), jnp.int32),
            pltpu.VMEM((eb,), jnp.int32),
            pltpu.VMEM((eb, CW), jnp.float32),
            pltpu.VMEM((ZB, CW), jnp.float32),
            pltpu.SemaphoreType.DMA,
        ],
    )
    def k(*refs):
        tabs = refs[:n_chunks]
        src_h, dst_h, out_h, acc, idxs, idxd, rows, zbuf, sem = refs[n_chunks:]
        cid = lax.axis_index("c")
        sid = lax.axis_index("s")

        def zb(i, _):
            zbuf[i, :] = jnp.zeros((L,), jnp.float32)
            return 0
        lax.fori_loop(0, ZB, zb, 0)

        base = sid * e_pt
        for c in range(n_chunks):
            on = (cid == (c % NC))

            @pl.when(on)
            def _():
                _zero_stripe(acc, zbuf, sid * stripe, stripe, ZB)
            plsc.subcore_barrier()

            @pl.when(on)
            def _():
                def body(b, _):
                    off = base + b * eb
                    pltpu.sync_copy(src_h.at[pl.ds(off, eb)], idxs)
                    pltpu.sync_copy(dst_h.at[pl.ds(off, eb)], idxd)
                    pltpu.async_copy(tabs[c].at[idxs], rows, sem).wait()
                    pltpu.sync_copy(rows, acc.at[idxd], add=True)
                    return 0
                lax.fori_loop(0, nb, body, 0)
            plsc.subcore_barrier()

            @pl.when(on)
            def _():
                _dump_stripe(acc, rows, out_h, sid * stripe,
                             c * n_out_pad + sid * stripe, stripe, eb)
            plsc.subcore_barrier()

    return k(*tables, src, dst)


def _translate_edges(cluster_pad, src, dst, trash, eb):
    n = cluster_pad.shape[0]
    e = src.shape[0]
    e_pw = e // NW
    nb = e_pw // eb

    @functools.partial(
        pl.kernel,
        out_type=(jax.ShapeDtypeStruct((e,), jnp.int32),
                  jax.ShapeDtypeStruct((e,), jnp.int32)),
        mesh=_mesh(),
        compiler_params=_SC_PARAMS,
        scratch_types=[
            pltpu.VMEM((n,), jnp.int32),
            pltpu.VMEM((eb,), jnp.int32),
            pltpu.VMEM((eb,), jnp.int32),
            pltpu.VMEM((eb,), jnp.int32),
            pltpu.VMEM((eb,), jnp.int32),
        ],
    )
    def k(cl_h, src_h, dst_h, nsrc_h, mdst_h, cl, idxs, idxd, obs, obd):
        wid = _wid()
        pltpu.sync_copy(cl_h, cl)
        base = wid * e_pw
        trash_v = jnp.full((L,), trash, jnp.int32)

        def body(b, _):
            off = base + b * eb
            pltpu.sync_copy(src_h.at[pl.ds(off, eb)], idxs)
            pltpu.sync_copy(dst_h.at[pl.ds(off, eb)], idxd)

            def inner(j, _):
                s16 = idxs[pl.ds(j * L, L)]
                d16 = idxd[pl.ds(j * L, L)]
                cs = plsc.load_gather(cl, [s16])
                cd = plsc.load_gather(cl, [d16])
                obs[pl.ds(j * L, L)] = cs
                obd[pl.ds(j * L, L)] = jnp.where(cs == cd, trash_v, cd)
                return 0

            lax.fori_loop(0, eb // L, inner, 0)
            pltpu.sync_copy(obs, nsrc_h.at[pl.ds(off, eb)])
            pltpu.sync_copy(obd, mdst_h.at[pl.ds(off, eb)])
            return 0

        lax.fori_loop(0, nb, body, 0)

    return k(cluster_pad, src, dst)


def _edge_dot(s_pad, src, dst, eb):
    n = s_pad.shape[0]
    e = src.shape[0]
    e_pw = e // NW
    nb = e_pw // eb

    @functools.partial(
        pl.kernel,
        out_type=jax.ShapeDtypeStruct((NW * L,), jnp.float32),
        mesh=_mesh(),
        compiler_params=_SC_PARAMS,
        scratch_types=[
            pltpu.VMEM((n,), jnp.float32),
            pltpu.VMEM((eb,), jnp.int32),
            pltpu.VMEM((eb,), jnp.int32),
            pltpu.VMEM((L,), jnp.float32),
        ],
    )
    def k(s_h, src_h, dst_h, out_h, sv, idxs, idxd, accb):
        wid = _wid()
        pltpu.sync_copy(s_h, sv)
        base = wid * e_pw

        def body(b, acc):
            off = base + b * eb
            pltpu.sync_copy(src_h.at[pl.ds(off, eb)], idxs)
            pltpu.sync_copy(dst_h.at[pl.ds(off, eb)], idxd)

            def inner(j, a):
                s16 = idxs[pl.ds(j * L, L)]
                d16 = idxd[pl.ds(j * L, L)]
                return a + (plsc.load_gather(sv, [s16])
                            * plsc.load_gather(sv, [d16]))

            return lax.fori_loop(0, eb // L, inner, acc)

        acc = lax.fori_loop(0, nb, body, jnp.zeros((L,), jnp.float32))
        accb[...] = acc
        pltpu.sync_copy(accb, out_h.at[pl.ds(wid * L, L)])

    return k(s_pad, src, dst)


def _rmw_max(acc, d16, v16, active):
    def cond(p):
        return jnp.any(p)

    def body(p):
        cur = plsc.load_gather(acc, [d16], mask=p)
        upd = p & (v16 > cur)
        plsc.store_scatter(acc, [d16], v16, mask=upd)
        chk = plsc.load_gather(acc, [d16], mask=upd)
        return upd & (chk < v16)

    lax.while_loop(cond, body, active)


def _hop_best(skey_pad, src, dst, eb):
    e = src.shape[0]
    e_pw = e // NW
    nb = e_pw // eb

    @functools.partial(
        pl.kernel,
        out_type=jax.ShapeDtypeStruct((NW * N_PAD,), jnp.float32),
        mesh=_mesh(),
        compiler_params=_SC_PARAMS,
        scratch_types=[
            pltpu.VMEM((N_PAD,), jnp.float32),
            pltpu.VMEM((eb,), jnp.int32),
            pltpu.VMEM((eb,), jnp.int32),
            pltpu.VMEM((eb,), jnp.float32),
            pltpu.SemaphoreType.DMA,
        ],
    )
    def k(skey_h, src_h, dst_h, out_h, acc, idxs, idxd, kv, sem):
        wid = _wid()
        _fill_1d(acc, N_PAD, NEG_INF, jnp.float32)
        base = wid * e_pw
        ones = jnp.ones((L,), jnp.bool_)

        def body(b, _):
            off = base + b * eb
            pltpu.sync_copy(src_h.at[pl.ds(off, eb)], idxs)
            pltpu.async_copy(skey_h.at[idxs], kv, sem).wait()
            pltpu.sync_copy(dst_h.at[pl.ds(off, eb)], idxd)

            def inner(j, _):
                d16 = idxd[pl.ds(j * L, L)]
                v16 = kv[pl.ds(j * L, L)]
                _rmw_max(acc, d16, v16, ones)
                return 0

            lax.fori_loop(0, eb // L, inner, 0)
            return 0

        lax.fori_loop(0, nb, body, 0)
        pltpu.sync_copy(acc, out_h.at[pl.ds(wid * N_PAD, N_PAD)])

    return k(skey_pad, src, dst)


def _hop_win(skey_pad, cluster_pad, best_pad, src, dst, eb):
    e = src.shape[0]
    e_pw = e // NW
    nb = e_pw // eb

    @functools.partial(
        pl.kernel,
        out_type=jax.ShapeDtypeStruct((NW * N_PAD,), jnp.int32),
        mesh=_mesh(),
        compiler_params=_SC_PARAMS,
        scratch_types=[
            pltpu.VMEM((N_PAD,), jnp.int32),
            pltpu.VMEM((eb,), jnp.int32),
            pltpu.VMEM((eb,), jnp.int32),
            pltpu.VMEM((eb,), jnp.float32),
            pltpu.VMEM((eb,), jnp.int32),
            pltpu.VMEM((eb,), jnp.float32),
            pltpu.SemaphoreType.DMA,
        ],
    )
    def k(skey_h, cl_h, best_h, src_h, dst_h, out_h,
          acc, idxs, idxd, kv, cv, bv, sem):
        wid = _wid()
        _fill_1d(acc, N_PAD, I32_MIN, jnp.int32)
        base = wid * e_pw
        ones = jnp.ones((L,), jnp.bool_)
        min_v = jnp.full((L,), I32_MIN, jnp.int32)

        def body(b, _):
            off = base + b * eb
            pltpu.sync_copy(src_h.at[pl.ds(off, eb)], idxs)
            pltpu.async_copy(skey_h.at[idxs], kv, sem).wait()
            pltpu.async_copy(cl_h.at[idxs], cv, sem).wait()
            pltpu.sync_copy(dst_h.at[pl.ds(off, eb)], idxd)
            pltpu.async_copy(best_h.at[idxd], bv, sem).wait()

            def inner(j, _):
                d16 = idxd[pl.ds(j * L, L)]
                k16 = kv[pl.ds(j * L, L)]
                c16 = cv[pl.ds(j * L, L)]
                b16 = bv[pl.ds(j * L, L)]
                v16 = jnp.where(k16 >= b16, c16, min_v)
                _rmw_max(acc, d16, v16, ones)
                return 0

            lax.fori_loop(0, eb // L, inner, 0)
            return 0

        lax.fori_loop(0, nb, body, 0)
        pltpu.sync_copy(acc, out_h.at[pl.ds(wid * N_PAD, N_PAD)])

    return k(skey_pad, cluster_pad, best_pad, src, dst)


NB = 2000


def _gin_body(x_ref, a0_ref, a1_ref, W1_ref, b1_ref, W2_ref, b2_ref, *out_refs):
    h0 = x_ref[...] + a0_ref[...] + a1_ref[...]
    h1 = jnp.maximum(jnp.dot(h0, W1_ref[...], preferred_element_type=jnp.float32)
                     + b1_ref[...], 0.0)
    h = jnp.dot(h1, W2_ref[...], preferred_element_type=jnp.float32) + b2_ref[...]
    for c, r in enumerate(out_refs):
        r[...] = h[:, c * CW:(c + 1) * CW]


def _gin_mlp_chunked(x, a0, a1, W1, b1, W2, b2, n_rows):
    grid = (n_rows // NB,)
    f_in = x.shape[1]
    f_mid = W1.shape[1]
    n_out_chunks = HID // CW
    return pl.pallas_call(
        _gin_body,
        grid=grid,
        in_specs=[
            pl.BlockSpec((NB, f_in), lambda i: (i, 0)),
            pl.BlockSpec((NB, f_in), lambda i: (i, 0)),
            pl.BlockSpec((NB, f_in), lambda i: (i, 0)),
            pl.BlockSpec((f_in, f_mid), lambda i: (0, 0)),
            pl.BlockSpec((1, f_mid), lambda i: (0, 0)),
            pl.BlockSpec((f_mid, HID), lambda i: (0, 0)),
            pl.BlockSpec((1, HID), lambda i: (0, 0)),
        ],
        out_specs=[pl.BlockSpec((NB, CW), lambda i: (i, 0))
                   for _ in range(n_out_chunks)],
        out_shape=[jax.ShapeDtypeStruct((n_rows, CW), jnp.float32)
                   for _ in range(n_out_chunks)],
    )(x, a0, a1, W1, b1.reshape(1, -1), W2, b2.reshape(1, -1))


def _score_body(h0_ref, h1_ref, h2_ref, h3_ref, g0_ref, g1_ref, g2_ref, g3_ref,
                deg_ref, Wmp_ref, bmp_ref, Wm1_ref, bm1_ref, Wm2_ref, bm2_ref,
                Ws_ref, bs_ref, s_ref, *hs_refs):
    h = jnp.concatenate([h0_ref[...], h1_ref[...], h2_ref[...], h3_ref[...]],
                        axis=1)
    agg = jnp.concatenate([g0_ref[...], g1_ref[...], g2_ref[...], g3_ref[...]],
                          axis=1)
    aggn = agg / jnp.maximum(deg_ref[...], 1.0)
    z = jnp.maximum(jnp.dot(h + aggn, Wmp_ref[...],
                            preferred_element_type=jnp.float32) + bmp_ref[...], 0.0)
    z = jnp.maximum(jnp.dot(z, Wm1_ref[...],
                            preferred_element_type=jnp.float32) + bm1_ref[...], 0.0)
    z = jnp.maximum(jnp.dot(z, Wm2_ref[...],
                            preferred_element_type=jnp.float32) + bm2_ref[...], 0.0)
    s = jnp.tanh(jnp.dot(z, Ws_ref[...],
                         preferred_element_type=jnp.float32) + bs_ref[...])
    s_ref[...] = s
    hs = h * s
    for c, r in enumerate(hs_refs):
        r[...] = hs[:, c * CW:(c + 1) * CW]


def _score_net(hc, aggc, deg, Wmp, bmp, Wm1, bm1, Wm2, bm2, Ws, bs):
    grid = (N_NODES // NB,)
    outs = pl.pallas_call(
        _score_body,
        grid=grid,
        in_specs=(
            [pl.BlockSpec((NB, CW), lambda i: (i, 0)) for _ in range(4)]
            + [pl.BlockSpec((NB, CW), lambda i: (i, 0)) for _ in range(4)]
            + [pl.BlockSpec((NB, 1), lambda i: (i, 0)),
               pl.BlockSpec((HID, HID), lambda i: (0, 0)),
               pl.BlockSpec((1, HID), lambda i: (0, 0)),
               pl.BlockSpec((HID, 32), lambda i: (0, 0)),
               pl.BlockSpec((1, 32), lambda i: (0, 0)),
               pl.BlockSpec((32, 32), lambda i: (0, 0)),
               pl.BlockSpec((1, 32), lambda i: (0, 0)),
               pl.BlockSpec((32, 1), lambda i: (0, 0)),
               pl.BlockSpec((1, 1), lambda i: (0, 0))]
        ),
        out_specs=[pl.BlockSpec((NB, 1), lambda i: (i, 0))]
        + [pl.BlockSpec((NB, CW), lambda i: (i, 0)) for _ in range(4)],
        out_shape=[jax.ShapeDtypeStruct((N_NODES, 1), jnp.float32)]
        + [jax.ShapeDtypeStruct((N_NODES, CW), jnp.float32) for _ in range(4)],
    )(*hc, *aggc, deg.reshape(-1, 1), Wmp, bmp.reshape(1, -1), Wm1,
      bm1.reshape(1, -1), Wm2, bm2.reshape(1, -1), Ws, bs.reshape(1, -1))
    return outs[0][:, 0], outs[1:]


def _logits_body(p0, p1, p2, p3, cnt_ref, Wl_ref, bl_ref, out_ref):
    pooled = jnp.concatenate([p0[...], p1[...], p2[...], p3[...]], axis=1)
    pooled = pooled / jnp.maximum(cnt_ref[...], 1.0)
    logits = jnp.dot(pooled, Wl_ref[...],
                     preferred_element_type=jnp.float32) + bl_ref[...]
    m = jnp.max(logits, axis=1, keepdims=True)
    sh = logits - m
    lse = jnp.log(jnp.sum(jnp.exp(sh), axis=1, keepdims=True))
    out_ref[...] = sh - lse


def _logits(pooled_chunks, cnt, Wl, bl):
    return pl.pallas_call(
        _logits_body,
        grid=(1,),
        in_specs=[pl.BlockSpec((G_PAD, CW), lambda i: (0, 0)) for _ in range(4)]
        + [pl.BlockSpec((G_PAD, 1), lambda i: (0, 0)),
           pl.BlockSpec((HID, N_CLASSES), lambda i: (0, 0)),
           pl.BlockSpec((1, N_CLASSES), lambda i: (0, 0))],
        out_specs=pl.BlockSpec((G_PAD, N_CLASSES), lambda i: (0, 0)),
        out_shape=jax.ShapeDtypeStruct((G_PAD, N_CLASSES), jnp.float32),
    )(*pooled_chunks, cnt, Wl, bl.reshape(1, -1))


def _pad_rows(a, n_pad):
    return jnp.pad(a, ((0, n_pad - a.shape[0]), (0, 0)))


def _pad_1d(a, n_pad, val=0):
    return jnp.pad(a, (0, n_pad - a.shape[0]), constant_values=val)


def kernel(x, edge_index, batch, W1, b1, W2, b2, Wmp, bmp, Wm1, bm1, Wm2, bm2,
           Ws, bs, V1, c1, V2, c2, Wl, bl):
    src, dst = edge_index[0], edge_index[1]
    n = x.shape[0]
    e = src.shape[0]

    x16 = jnp.concatenate(
        [x, jnp.ones((n, 1), jnp.float32), jnp.zeros((n, CW - F_IN - 1), jnp.float32)],
        axis=1)
    x16p = _pad_rows(x16, N_PAD)
    agg1p = _seg_sum_partial(x16p, src, dst, N_PAD, 1000)
    agg1p = agg1p.reshape(NC, N_PAD, CW)
    a0 = agg1p[0, :n, :]
    a1 = agg1p[1, :n, :]
    deg = a0[:, F_IN] + a1[:, F_IN]
    W1p = jnp.pad(W1, ((0, CW - F_IN), (0, 0)))
    hc = _gin_mlp_chunked(x16, a0, a1, W1p, b1, W2, b2, n)

    hcp = [_pad_rows(h, N_PAD) for h in hc]
    aggh = _seg_sum_chunks(hcp, src, dst, N_PAD, 1000).reshape(4, N_PAD, CW)
    aggc = [aggh[c, :n, :] for c in range(4)]
    s, hs_chunks = _score_net(hc, aggc, deg, Wmp, bmp, Wm1, bm1, Wm2, bm2, Ws, bs)

    sp = _pad_1d(s, N_PAD)
    mc_parts = _edge_dot(sp, src, dst, 2000)
    mc_loss = BETA * jnp.sum(mc_parts) / jnp.float32(e)

    _, idx = jax.lax.top_k(s, K_POOL)
    cluster = jnp.full((n,), -1, jnp.int32).at[idx].set(
        jnp.arange(K_POOL, dtype=jnp.int32))

    for _ in range(N_HOPS):
        skey = jnp.where(cluster >= 0, s, NEG_INF)
        skp = _pad_1d(skey, N_PAD, NEG_INF)
        clp = _pad_1d(cluster, N_PAD, -1)
        bp = _hop_best(skp, src, dst, 2000).reshape(NW, N_PAD)
        best = jnp.max(bp, axis=0)
        wp = _hop_win(skp, clp, best, src, dst, 2000).reshape(NW, N_PAD)
        win = jnp.max(wp, axis=0)[:n]
        cluster = jnp.where(cluster >= 0, cluster, jnp.maximum(win, -1))
    cluster = jnp.where(cluster >= 0, cluster, 0)

    E_XP = N_PAD
    hs_pad = [_pad_rows(c, N_PAD) for c in hs_chunks]
    lin_idx = jnp.arange(E_XP, dtype=jnp.int32)
    xp_dst = _pad_1d(cluster, E_XP, K_POOL)
    x_pool = _seg_sum_chunks(hs_pad, lin_idx, xp_dst, K_PAD, 3128)
    x_pool = x_pool.reshape(4, K_PAD, CW)
    batch_pool = batch[idx]

    clp = _pad_1d(cluster, N_PAD, 0)
    nsrc, mdst = _translate_edges(clp, src, dst, K_POOL, 2000)
    xp_chunks = [x_pool[c] for c in range(4)]
    agg2 = _seg_sum_chunks(xp_chunks, nsrc, mdst, K_PAD, 1000)
    agg2 = agg2.reshape(4, K_PAD, CW)
    xpc = jnp.concatenate([x_pool[c, :K_POOL, :] for c in range(4)], axis=1)
    a2 = jnp.concatenate([agg2[c, :K_POOL, :] for c in range(4)], axis=1)
    zeros_k = jnp.zeros((K_POOL, HID), jnp.float32)
    h2c = _gin_mlp_chunked(xpc, a2, zeros_k, V1, c1, V2, c2, K_POOL)

    h2p = [_pad_rows(c, K_PAD) for c in h2c]
    ones_chunk = jnp.zeros((K_PAD, CW), jnp.float32).at[:K_POOL, 0].set(1.0)
    lin_k = jnp.arange(K_PAD, dtype=jnp.int32)
    ro_dst = _pad_1d(batch_pool, K_PAD, N_GRAPHS)
    ro = _seg_sum_chunks(h2p + [ones_chunk], lin_k, ro_dst, G_PAD, 3128)
    ro = ro.reshape(5, G_PAD, CW)
    pooled_chunks = [ro[c] for c in range(4)]
    cnt = ro[4, :, 0:1]
    out = _logits(pooled_chunks, cnt, Wl, bl)
    return out[:N_GRAPHS], mc_loss


# speedup vs baseline: 22.8241x; 1.0035x over previous
import functools

import jax
import jax.numpy as jnp
from jax import lax
from jax.experimental import pallas as pl
from jax.experimental.pallas import tpu as pltpu
from jax.experimental.pallas import tpu_sc as plsc

N_NODES = 100000
N_EDGES = 1600000
F_IN = 7
HID = 64
N_CLASSES = 2
N_GRAPHS = 20
RATIO = 0.5
BETA = 1.0
K_POOL = int(N_NODES * RATIO)
N_HOPS = 3

NC, NS, L = 2, 16, 16
NW = NC * NS
CW = 16

N_PAD = 100096
K_PAD = 50048
G_PAD = 32
NEG_INF = float("-inf")
I32_MIN = -2147483648
ZB = 208

_mesh = functools.partial(plsc.VectorSubcoreMesh,
                          core_axis_name="c", subcore_axis_name="s")
_SC_PARAMS = pltpu.CompilerParams(use_tc_tiling_on_sc=False, needs_layout_passes=False)


def _wid():
    return lax.axis_index("s") * NC + lax.axis_index("c")


def _fill_1d(ref, n, val, dtype):
    v = jnp.full((L,), val, dtype)

    def body(i, _):
        ref[pl.ds(i * L, L)] = v
        return 0

    lax.fori_loop(0, n // L, body, 0)


def _zero_stripe(acc, zbuf, row0, nrows, zb):
    full, rem = nrows // zb, nrows % zb
    for k in range(full):
        pltpu.sync_copy(zbuf, acc.at[pl.ds(row0 + k * zb, zb)])
    if rem:
        pltpu.sync_copy(zbuf.at[pl.ds(0, rem)],
                        acc.at[pl.ds(row0 + full * zb, rem)])


def _dump_stripe(acc, bounce, out_hbm, src_row0, dst_row0, nrows, bb):
    full, rem = nrows // bb, nrows % bb
    for k in range(full):
        pltpu.sync_copy(acc.at[pl.ds(src_row0 + k * bb, bb)], bounce)
        pltpu.sync_copy(bounce, out_hbm.at[pl.ds(dst_row0 + k * bb, bb)])
    if rem:
        pltpu.sync_copy(acc.at[pl.ds(src_row0 + full * bb, rem)],
                        bounce.at[pl.ds(0, rem)])
        pltpu.sync_copy(bounce.at[pl.ds(0, rem)],
                        out_hbm.at[pl.ds(dst_row0 + full * bb, rem)])


def _seg_sum_partial(table, src, dst, n_out_pad, eb):
    e = src.shape[0]
    e_pw = e // NW
    nb = e_pw // eb
    stripe = n_out_pad // NS

    @functools.partial(
        pl.kernel,
        out_type=jax.ShapeDtypeStruct((NC * n_out_pad, CW), jnp.float32),
        mesh=_mesh(),
        compiler_params=_SC_PARAMS,
        scratch_types=[
            pltpu.VMEM_SHARED((n_out_pad, CW), jnp.float32),
            pltpu.VMEM((eb,), jnp.int32),
            pltpu.VMEM((eb,), jnp.int32),
            pltpu.VMEM((eb, CW), jnp.float32),
            pltpu.SemaphoreType.DMA,
        ],
    )
    def k(table_h, src_h, dst_h, out_h, acc, idxs, idxd, rows, sem):
        cid = lax.axis_index("c")
        sid = lax.axis_index("s")
        wid = sid * NC + cid
        def zb(i, _):
            rows[i, :] = jnp.zeros((L,), jnp.float32)
            return 0
        lax.fori_loop(0, eb, zb, 0)
        _zero_stripe(acc, rows, sid * stripe, stripe, eb)
        plsc.subcore_barrier()

        base = wid * e_pw

        def body(b, _):
            off = base + b * eb
            pltpu.sync_copy(src_h.at[pl.ds(off, eb)], idxs)
            pltpu.sync_copy(dst_h.at[pl.ds(off, eb)], idxd)
            pltpu.async_copy(table_h.at[idxs], rows, sem).wait()
            pltpu.sync_copy(rows, acc.at[idxd], add=True)
            return 0

        lax.fori_loop(0, nb, body, 0)
        plsc.subcore_barrier()
        _dump_stripe(acc, rows, out_h, sid * stripe,
                     cid * n_out_pad + sid * stripe, stripe, eb)

    return k(table, src, dst)


def _seg_sum_chunks(tables, src, dst, n_out_pad, eb):
    n_chunks = len(tables)
    e = src.shape[0]
    e_pt = e // NS
    nb = e_pt // eb
    stripe = n_out_pad // NS

    @functools.partial(
        pl.kernel,
        out_type=jax.ShapeDtypeStruct((n_chunks * n_out_pad, CW), jnp.float32),
        mesh=_mesh(),
        compiler_params=_SC_PARAMS,
        scratch_types=[
            pltpu.VMEM_SHARED((n_out_pad, CW), jnp.float32),
            pltpu.VMEM((eb,), jnp.int32),
            pltpu.VMEM((eb,), jnp.int32),
            pltpu.VMEM((eb, CW), jnp.float32),
            pltpu.VMEM((ZB, CW), jnp.float32),
            pltpu.SemaphoreType.DMA,
        ],
    )
    def k(*refs):
        tabs = refs[:n_chunks]
        src_h, dst_h, out_h, acc, idxs, idxd, rows, zbuf, sem = refs[n_chunks:]
        cid = lax.axis_index("c")
        sid = lax.axis_index("s")

        def zb(i, _):
            zbuf[i, :] = jnp.zeros((L,), jnp.float32)
            return 0
        lax.fori_loop(0, ZB, zb, 0)

        base = sid * e_pt
        for c in range(n_chunks):
            on = (cid == (c % NC))

            @pl.when(on)
            def _():
                _zero_stripe(acc, zbuf, sid * stripe, stripe, ZB)
            plsc.subcore_barrier()

            @pl.when(on)
            def _():
                def body(b, _):
                    off = base + b * eb
                    pltpu.sync_copy(src_h.at[pl.ds(off, eb)], idxs)
                    pltpu.sync_copy(dst_h.at[pl.ds(off, eb)], idxd)
                    pltpu.async_copy(tabs[c].at[idxs], rows, sem).wait()
                    pltpu.sync_copy(rows, acc.at[idxd], add=True)
                    return 0
                lax.fori_loop(0, nb, body, 0)
            plsc.subcore_barrier()

            @pl.when(on)
            def _():
                _dump_stripe(acc, rows, out_h, sid * stripe,
                             c * n_out_pad + sid * stripe, stripe, eb)
            plsc.subcore_barrier()

    return k(*tables, src, dst)


def _translate_edges(cluster_pad, src, dst, trash, eb):
    n = cluster_pad.shape[0]
    e = src.shape[0]
    e_pw = e // NW
    nb = e_pw // eb

    @functools.partial(
        pl.kernel,
        out_type=(jax.ShapeDtypeStruct((e,), jnp.int32),
                  jax.ShapeDtypeStruct((e,), jnp.int32)),
        mesh=_mesh(),
        compiler_params=_SC_PARAMS,
        scratch_types=[
            pltpu.VMEM((n,), jnp.int32),
            pltpu.VMEM((eb,), jnp.int32),
            pltpu.VMEM((eb,), jnp.int32),
            pltpu.VMEM((eb,), jnp.int32),
            pltpu.VMEM((eb,), jnp.int32),
        ],
    )
    def k(cl_h, src_h, dst_h, nsrc_h, mdst_h, cl, idxs, idxd, obs, obd):
        wid = _wid()
        pltpu.sync_copy(cl_h, cl)
        base = wid * e_pw
        trash_v = jnp.full((L,), trash, jnp.int32)

        def body(b, _):
            off = base + b * eb
            pltpu.sync_copy(src_h.at[pl.ds(off, eb)], idxs)
            pltpu.sync_copy(dst_h.at[pl.ds(off, eb)], idxd)

            def inner(j, _):
                s16 = idxs[pl.ds(j * L, L)]
                d16 = idxd[pl.ds(j * L, L)]
                cs = plsc.load_gather(cl, [s16])
                cd = plsc.load_gather(cl, [d16])
                obs[pl.ds(j * L, L)] = cs
                obd[pl.ds(j * L, L)] = jnp.where(cs == cd, trash_v, cd)
                return 0

            lax.fori_loop(0, eb // L, inner, 0)
            pltpu.sync_copy(obs, nsrc_h.at[pl.ds(off, eb)])
            pltpu.sync_copy(obd, mdst_h.at[pl.ds(off, eb)])
            return 0

        lax.fori_loop(0, nb, body, 0)

    return k(cluster_pad, src, dst)


def _edge_dot(s_pad, src, dst, eb):
    n = s_pad.shape[0]
    e = src.shape[0]
    e_pw = e // NW
    nb = e_pw // eb

    @functools.partial(
        pl.kernel,
        out_type=jax.ShapeDtypeStruct((NW * L,), jnp.float32),
        mesh=_mesh(),
        compiler_params=_SC_PARAMS,
        scratch_types=[
            pltpu.VMEM((n,), jnp.float32),
            pltpu.VMEM((eb,), jnp.int32),
            pltpu.VMEM((eb,), jnp.int32),
            pltpu.VMEM((L,), jnp.float32),
        ],
    )
    def k(s_h, src_h, dst_h, out_h, sv, idxs, idxd, accb):
        wid = _wid()
        pltpu.sync_copy(s_h, sv)
        base = wid * e_pw

        def body(b, acc):
            off = base + b * eb
            pltpu.sync_copy(src_h.at[pl.ds(off, eb)], idxs)
            pltpu.sync_copy(dst_h.at[pl.ds(off, eb)], idxd)

            def inner(j, a):
                s16 = idxs[pl.ds(j * L, L)]
                d16 = idxd[pl.ds(j * L, L)]
                return a + (plsc.load_gather(sv, [s16])
                            * plsc.load_gather(sv, [d16]))

            return lax.fori_loop(0, eb // L, inner, acc)

        acc = lax.fori_loop(0, nb, body, jnp.zeros((L,), jnp.float32))
        accb[...] = acc
        pltpu.sync_copy(accb, out_h.at[pl.ds(wid * L, L)])

    return k(s_pad, src, dst)


def _rmw_max(acc, d16, v16, active):
    def cond(p):
        return jnp.any(p)

    def body(p):
        cur = plsc.load_gather(acc, [d16], mask=p)
        upd = p & (v16 > cur)
        plsc.store_scatter(acc, [d16], v16, mask=upd)
        chk = plsc.load_gather(acc, [d16], mask=upd)
        return upd & (chk < v16)

    lax.while_loop(cond, body, active)


def _hop_best(skey_pad, src, dst, eb):
    e = src.shape[0]
    e_pw = e // NW
    nb = e_pw // eb

    @functools.partial(
        pl.kernel,
        out_type=jax.ShapeDtypeStruct((NW * N_PAD,), jnp.float32),
        mesh=_mesh(),
        compiler_params=_SC_PARAMS,
        scratch_types=[
            pltpu.VMEM((N_PAD,), jnp.float32),
            pltpu.VMEM((eb,), jnp.int32),
            pltpu.VMEM((eb,), jnp.int32),
            pltpu.VMEM((eb,), jnp.float32),
            pltpu.SemaphoreType.DMA,
        ],
    )
    def k(skey_h, src_h, dst_h, out_h, acc, idxs, idxd, kv, sem):
        wid = _wid()
        _fill_1d(acc, N_PAD, NEG_INF, jnp.float32)
        base = wid * e_pw
        ones = jnp.ones((L,), jnp.bool_)

        def body(b, _):
            off = base + b * eb
            pltpu.sync_copy(src_h.at[pl.ds(off, eb)], idxs)
            pltpu.async_copy(skey_h.at[idxs], kv, sem).wait()
            pltpu.sync_copy(dst_h.at[pl.ds(off, eb)], idxd)

            def inner(j, _):
                d16 = idxd[pl.ds(j * L, L)]
                v16 = kv[pl.ds(j * L, L)]
                _rmw_max(acc, d16, v16, ones)
                return 0

            lax.fori_loop(0, eb // L, inner, 0)
            return 0

        lax.fori_loop(0, nb, body, 0)
        pltpu.sync_copy(acc, out_h.at[pl.ds(wid * N_PAD, N_PAD)])

    return k(skey_pad, src, dst)


def _hop_win(skey_pad, cluster_pad, best_pad, src, dst, eb):
    e = src.shape[0]
    e_pw = e // NW
    nb = e_pw // eb

    @functools.partial(
        pl.kernel,
        out_type=jax.ShapeDtypeStruct((NW * N_PAD,), jnp.int32),
        mesh=_mesh(),
        compiler_params=_SC_PARAMS,
        scratch_types=[
            pltpu.VMEM((N_PAD,), jnp.int32),
            pltpu.VMEM((eb,), jnp.int32),
            pltpu.VMEM((eb,), jnp.int32),
            pltpu.VMEM((eb,), jnp.float32),
            pltpu.VMEM((eb,), jnp.int32),
            pltpu.VMEM((eb,), jnp.float32),
            pltpu.SemaphoreType.DMA,
        ],
    )
    def k(skey_h, cl_h, best_h, src_h, dst_h, out_h,
          acc, idxs, idxd, kv, cv, bv, sem):
        wid = _wid()
        _fill_1d(acc, N_PAD, I32_MIN, jnp.int32)
        base = wid * e_pw
        ones = jnp.ones((L,), jnp.bool_)
        min_v = jnp.full((L,), I32_MIN, jnp.int32)

        def body(b, _):
            off = base + b * eb
            pltpu.sync_copy(src_h.at[pl.ds(off, eb)], idxs)
            pltpu.async_copy(skey_h.at[idxs], kv, sem).wait()
            pltpu.async_copy(cl_h.at[idxs], cv, sem).wait()
            pltpu.sync_copy(dst_h.at[pl.ds(off, eb)], idxd)
            pltpu.async_copy(best_h.at[idxd], bv, sem).wait()

            def inner(j, _):
                d16 = idxd[pl.ds(j * L, L)]
                k16 = kv[pl.ds(j * L, L)]
                c16 = cv[pl.ds(j * L, L)]
                b16 = bv[pl.ds(j * L, L)]
                v16 = jnp.where(k16 >= b16, c16, min_v)
                _rmw_max(acc, d16, v16, ones)
                return 0

            lax.fori_loop(0, eb // L, inner, 0)
            return 0

        lax.fori_loop(0, nb, body, 0)
        pltpu.sync_copy(acc, out_h.at[pl.ds(wid * N_PAD, N_PAD)])

    return k(skey_pad, cluster_pad, best_pad, src, dst)


NB = 2000


def _gin_body(x_ref, a0_ref, a1_ref, W1_ref, b1_ref, W2_ref, b2_ref, *out_refs):
    h0 = x_ref[...] + a0_ref[...] + a1_ref[...]
    h1 = jnp.maximum(jnp.dot(h0, W1_ref[...], preferred_element_type=jnp.float32)
                     + b1_ref[...], 0.0)
    h = jnp.dot(h1, W2_ref[...], preferred_element_type=jnp.float32) + b2_ref[...]
    for c, r in enumerate(out_refs):
        r[...] = h[:, c * CW:(c + 1) * CW]


def _gin_mlp_chunked(x, a0, a1, W1, b1, W2, b2, n_rows):
    grid = (n_rows // NB,)
    f_in = x.shape[1]
    f_mid = W1.shape[1]
    n_out_chunks = HID // CW
    return pl.pallas_call(
        _gin_body,
        grid=grid,
        in_specs=[
            pl.BlockSpec((NB, f_in), lambda i: (i, 0)),
            pl.BlockSpec((NB, f_in), lambda i: (i, 0)),
            pl.BlockSpec((NB, f_in), lambda i: (i, 0)),
            pl.BlockSpec((f_in, f_mid), lambda i: (0, 0)),
            pl.BlockSpec((1, f_mid), lambda i: (0, 0)),
            pl.BlockSpec((f_mid, HID), lambda i: (0, 0)),
            pl.BlockSpec((1, HID), lambda i: (0, 0)),
        ],
        out_specs=[pl.BlockSpec((NB, CW), lambda i: (i, 0))
                   for _ in range(n_out_chunks)],
        out_shape=[jax.ShapeDtypeStruct((n_rows, CW), jnp.float32)
                   for _ in range(n_out_chunks)],
    )(x, a0, a1, W1, b1.reshape(1, -1), W2, b2.reshape(1, -1))


def _score_body(h0_ref, h1_ref, h2_ref, h3_ref, g0_ref, g1_ref, g2_ref, g3_ref,
                deg_ref, Wmp_ref, bmp_ref, Wm1_ref, bm1_ref, Wm2_ref, bm2_ref,
                Ws_ref, bs_ref, s_ref, *hs_refs):
    h = jnp.concatenate([h0_ref[...], h1_ref[...], h2_ref[...], h3_ref[...]],
                        axis=1)
    agg = jnp.concatenate([g0_ref[...], g1_ref[...], g2_ref[...], g3_ref[...]],
                          axis=1)
    aggn = agg / jnp.maximum(deg_ref[...], 1.0)
    z = jnp.maximum(jnp.dot(h + aggn, Wmp_ref[...],
                            preferred_element_type=jnp.float32) + bmp_ref[...], 0.0)
    z = jnp.maximum(jnp.dot(z, Wm1_ref[...],
                            preferred_element_type=jnp.float32) + bm1_ref[...], 0.0)
    z = jnp.maximum(jnp.dot(z, Wm2_ref[...],
                            preferred_element_type=jnp.float32) + bm2_ref[...], 0.0)
    s = jnp.tanh(jnp.dot(z, Ws_ref[...],
                         preferred_element_type=jnp.float32) + bs_ref[...])
    s_ref[...] = s
    hs = h * s
    for c, r in enumerate(hs_refs):
        r[...] = hs[:, c * CW:(c + 1) * CW]


def _score_net(hc, aggc, deg, Wmp, bmp, Wm1, bm1, Wm2, bm2, Ws, bs):
    grid = (N_NODES // NB,)
    outs = pl.pallas_call(
        _score_body,
        grid=grid,
        in_specs=(
            [pl.BlockSpec((NB, CW), lambda i: (i, 0)) for _ in range(4)]
            + [pl.BlockSpec((NB, CW), lambda i: (i, 0)) for _ in range(4)]
            + [pl.BlockSpec((NB, 1), lambda i: (i, 0)),
               pl.BlockSpec((HID, HID), lambda i: (0, 0)),
               pl.BlockSpec((1, HID), lambda i: (0, 0)),
               pl.BlockSpec((HID, 32), lambda i: (0, 0)),
               pl.BlockSpec((1, 32), lambda i: (0, 0)),
               pl.BlockSpec((32, 32), lambda i: (0, 0)),
               pl.BlockSpec((1, 32), lambda i: (0, 0)),
               pl.BlockSpec((32, 1), lambda i: (0, 0)),
               pl.BlockSpec((1, 1), lambda i: (0, 0))]
        ),
        out_specs=[pl.BlockSpec((NB, 1), lambda i: (i, 0))]
        + [pl.BlockSpec((NB, CW), lambda i: (i, 0)) for _ in range(4)],
        out_shape=[jax.ShapeDtypeStruct((N_NODES, 1), jnp.float32)]
        + [jax.ShapeDtypeStruct((N_NODES, CW), jnp.float32) for _ in range(4)],
    )(*hc, *aggc, deg.reshape(-1, 1), Wmp, bmp.reshape(1, -1), Wm1,
      bm1.reshape(1, -1), Wm2, bm2.reshape(1, -1), Ws, bs.reshape(1, -1))
    return outs[0][:, 0], outs[1:]


def _logits_body(p0, p1, p2, p3, cnt_ref, Wl_ref, bl_ref, out_ref):
    pooled = jnp.concatenate([p0[...], p1[...], p2[...], p3[...]], axis=1)
    pooled = pooled / jnp.maximum(cnt_ref[...], 1.0)
    logits = jnp.dot(pooled, Wl_ref[...],
                     preferred_element_type=jnp.float32) + bl_ref[...]
    m = jnp.max(logits, axis=1, keepdims=True)
    sh = logits - m
    lse = jnp.log(jnp.sum(jnp.exp(sh), axis=1, keepdims=True))
    out_ref[...] = sh - lse


def _logits(pooled_chunks, cnt, Wl, bl):
    return pl.pallas_call(
        _logits_body,
        grid=(1,),
        in_specs=[pl.BlockSpec((G_PAD, CW), lambda i: (0, 0)) for _ in range(4)]
        + [pl.BlockSpec((G_PAD, 1), lambda i: (0, 0)),
           pl.BlockSpec((HID, N_CLASSES), lambda i: (0, 0)),
           pl.BlockSpec((1, N_CLASSES), lambda i: (0, 0))],
        out_specs=pl.BlockSpec((G_PAD, N_CLASSES), lambda i: (0, 0)),
        out_shape=jax.ShapeDtypeStruct((G_PAD, N_CLASSES), jnp.float32),
    )(*pooled_chunks, cnt, Wl, bl.reshape(1, -1))


def _pad_rows(a, n_pad):
    return jnp.pad(a, ((0, n_pad - a.shape[0]), (0, 0)))


def _pad_1d(a, n_pad, val=0):
    return jnp.pad(a, (0, n_pad - a.shape[0]), constant_values=val)


def kernel(x, edge_index, batch, W1, b1, W2, b2, Wmp, bmp, Wm1, bm1, Wm2, bm2,
           Ws, bs, V1, c1, V2, c2, Wl, bl):
    src, dst = edge_index[0], edge_index[1]
    n = x.shape[0]
    e = src.shape[0]

    x16 = jnp.concatenate(
        [x, jnp.ones((n, 1), jnp.float32), jnp.zeros((n, CW - F_IN - 1), jnp.float32)],
        axis=1)
    x16p = _pad_rows(x16, N_PAD)
    agg1p = _seg_sum_partial(x16p, src, dst, N_PAD, 1000)
    agg1p = agg1p.reshape(NC, N_PAD, CW)
    a0 = agg1p[0, :n, :]
    a1 = agg1p[1, :n, :]
    deg = a0[:, F_IN] + a1[:, F_IN]
    W1p = jnp.pad(W1, ((0, CW - F_IN), (0, 0)))
    hc = _gin_mlp_chunked(x16, a0, a1, W1p, b1, W2, b2, n)

    hcp = [_pad_rows(h, N_PAD) for h in hc]
    aggh = _seg_sum_chunks(hcp, src, dst, N_PAD, 1000).reshape(4, N_PAD, CW)
    aggc = [aggh[c, :n, :] for c in range(4)]
    s, hs_chunks = _score_net(hc, aggc, deg, Wmp, bmp, Wm1, bm1, Wm2, bm2, Ws, bs)

    sp = _pad_1d(s, N_PAD)
    mc_parts = _edge_dot(sp, src, dst, 2000)
    mc_loss = BETA * jnp.sum(mc_parts) / jnp.float32(e)

    u = lax.bitcast_convert_type(s, jnp.uint32)
    flip = jnp.where(u >> 31 == 1, jnp.uint32(0xFFFFFFFF), jnp.uint32(0x80000000))
    desc = ~(u ^ flip)
    _, order = lax.sort((desc, jnp.arange(n, dtype=jnp.int32)), num_keys=1)
    idx = order[:K_POOL]
    cluster = jnp.full((n,), -1, jnp.int32).at[idx].set(
        jnp.arange(K_POOL, dtype=jnp.int32))

    for _ in range(N_HOPS):
        skey = jnp.where(cluster >= 0, s, NEG_INF)
        skp = _pad_1d(skey, N_PAD, NEG_INF)
        clp = _pad_1d(cluster, N_PAD, -1)
        bp = _hop_best(skp, src, dst, 2000).reshape(NW, N_PAD)
        best = jnp.max(bp, axis=0)
        wp = _hop_win(skp, clp, best, src, dst, 2000).reshape(NW, N_PAD)
        win = jnp.max(wp, axis=0)[:n]
        cluster = jnp.where(cluster >= 0, cluster, jnp.maximum(win, -1))
    cluster = jnp.where(cluster >= 0, cluster, 0)

    E_XP = N_PAD
    hs_pad = [_pad_rows(c, N_PAD) for c in hs_chunks]
    lin_idx = jnp.arange(E_XP, dtype=jnp.int32)
    xp_dst = _pad_1d(cluster, E_XP, K_POOL)
    x_pool = _seg_sum_chunks(hs_pad, lin_idx, xp_dst, K_PAD, 3128)
    x_pool = x_pool.reshape(4, K_PAD, CW)
    batch_pool = batch[idx]

    clp = _pad_1d(cluster, N_PAD, 0)
    nsrc, mdst = _translate_edges(clp, src, dst, K_POOL, 2000)
    xp_chunks = [x_pool[c] for c in range(4)]
    agg2 = _seg_sum_chunks(xp_chunks, nsrc, mdst, K_PAD, 1000)
    agg2 = agg2.reshape(4, K_PAD, CW)
    xpc = jnp.concatenate([x_pool[c, :K_POOL, :] for c in range(4)], axis=1)
    a2 = jnp.concatenate([agg2[c, :K_POOL, :] for c in range(4)], axis=1)
    zeros_k = jnp.zeros((K_POOL, HID), jnp.float32)
    h2c = _gin_mlp_chunked(xpc, a2, zeros_k, V1, c1, V2, c2, K_POOL)

    h2p = [_pad_rows(c, K_PAD) for c in h2c]
    ones_chunk = jnp.zeros((K_PAD, CW), jnp.float32).at[:K_POOL, 0].set(1.0)
    lin_k = jnp.arange(K_PAD, dtype=jnp.int32)
    ro_dst = _pad_1d(batch_pool, K_PAD, N_GRAPHS)
    ro = _seg_sum_chunks(h2p + [ones_chunk], lin_k, ro_dst, G_PAD, 3128)
    ro = ro.reshape(5, G_PAD, CW)
    pooled_chunks = [ro[c] for c in range(4)]
    cnt = ro[4, :, 0:1]
    out = _logits(pooled_chunks, cnt, Wl, bl)
    return out[:N_GRAPHS], mc_loss


# speedup vs baseline: 24.5301x; 1.0747x over previous
import functools

import jax
import jax.numpy as jnp
from jax import lax
from jax.experimental import pallas as pl
from jax.experimental.pallas import tpu as pltpu
from jax.experimental.pallas import tpu_sc as plsc

N_NODES = 100000
N_EDGES = 1600000
F_IN = 7
HID = 64
N_CLASSES = 2
N_GRAPHS = 20
RATIO = 0.5
BETA = 1.0
K_POOL = int(N_NODES * RATIO)
N_HOPS = 3

NC, NS, L = 2, 16, 16
NW = NC * NS
CW = 16

N_PAD = 100096
K_PAD = 50048
G_PAD = 32
NEG_INF = float("-inf")
I32_MIN = -2147483648
ZB = 104

_mesh = functools.partial(plsc.VectorSubcoreMesh,
                          core_axis_name="c", subcore_axis_name="s")
_SC_PARAMS = pltpu.CompilerParams(use_tc_tiling_on_sc=False, needs_layout_passes=False)


def _wid():
    return lax.axis_index("s") * NC + lax.axis_index("c")


def _fill_1d(ref, n, val, dtype):
    v = jnp.full((L,), val, dtype)

    def body(i, _):
        ref[pl.ds(i * L, L)] = v
        return 0

    lax.fori_loop(0, n // L, body, 0)


def _zero_stripe(acc, zbuf, row0, nrows, zb):
    full, rem = nrows // zb, nrows % zb
    for k in range(full):
        pltpu.sync_copy(zbuf, acc.at[pl.ds(row0 + k * zb, zb)])
    if rem:
        pltpu.sync_copy(zbuf.at[pl.ds(0, rem)],
                        acc.at[pl.ds(row0 + full * zb, rem)])


def _dump_stripe(acc, bounce, out_hbm, src_row0, dst_row0, nrows, bb):
    full, rem = nrows // bb, nrows % bb
    for k in range(full):
        pltpu.sync_copy(acc.at[pl.ds(src_row0 + k * bb, bb)], bounce)
        pltpu.sync_copy(bounce, out_hbm.at[pl.ds(dst_row0 + k * bb, bb)])
    if rem:
        pltpu.sync_copy(acc.at[pl.ds(src_row0 + full * bb, rem)],
                        bounce.at[pl.ds(0, rem)])
        pltpu.sync_copy(bounce.at[pl.ds(0, rem)],
                        out_hbm.at[pl.ds(dst_row0 + full * bb, rem)])


def _seg_sum_partial(table, src, dst, n_out_pad, eb):
    e = src.shape[0]
    e_pw = e // NW
    nb = e_pw // eb
    stripe = n_out_pad // NS

    @functools.partial(
        pl.kernel,
        out_type=jax.ShapeDtypeStruct((NC * n_out_pad, CW), jnp.float32),
        mesh=_mesh(),
        compiler_params=_SC_PARAMS,
        scratch_types=[
            pltpu.VMEM_SHARED((n_out_pad, CW), jnp.float32),
            pltpu.VMEM((eb,), jnp.int32),
            pltpu.VMEM((eb,), jnp.int32),
            pltpu.VMEM((eb, CW), jnp.float32),
            pltpu.SemaphoreType.DMA,
        ],
    )
    def k(table_h, src_h, dst_h, out_h, acc, idxs, idxd, rows, sem):
        cid = lax.axis_index("c")
        sid = lax.axis_index("s")
        wid = sid * NC + cid
        def zb(i, _):
            rows[i, :] = jnp.zeros((L,), jnp.float32)
            return 0
        lax.fori_loop(0, eb, zb, 0)
        _zero_stripe(acc, rows, sid * stripe, stripe, eb)
        plsc.subcore_barrier()

        base = wid * e_pw

        def body(b, _):
            off = base + b * eb
            pltpu.sync_copy(src_h.at[pl.ds(off, eb)], idxs)
            pltpu.sync_copy(dst_h.at[pl.ds(off, eb)], idxd)
            pltpu.async_copy(table_h.at[idxs], rows, sem).wait()
            pltpu.sync_copy(rows, acc.at[idxd], add=True)
            return 0

        lax.fori_loop(0, nb, body, 0)
        plsc.subcore_barrier()
        _dump_stripe(acc, rows, out_h, sid * stripe,
                     cid * n_out_pad + sid * stripe, stripe, eb)

    return k(table, src, dst)


def _seg_sum_chunks(tables, src, dst, n_out_pad, eb):
    n_chunks = len(tables)
    e = src.shape[0]
    e_pt = e // NS
    nb = e_pt // eb
    stripe = n_out_pad // NS
    pipe = nb >= 4
    n_pairs, tail = (nb // 2, nb % 2) if pipe else (0, 0)

    scratch = [
        pltpu.VMEM_SHARED((n_out_pad, CW), jnp.float32),
        pltpu.VMEM((eb,), jnp.int32),
        pltpu.VMEM((eb,), jnp.int32),
        pltpu.VMEM((eb, CW), jnp.float32),
        pltpu.VMEM((ZB, CW), jnp.float32),
        pltpu.SemaphoreType.DMA,
    ]
    if pipe:
        scratch += [
            pltpu.VMEM((eb,), jnp.int32),
            pltpu.VMEM((eb,), jnp.int32),
            pltpu.VMEM((eb, CW), jnp.float32),
            pltpu.SemaphoreType.DMA,
        ]

    @functools.partial(
        pl.kernel,
        out_type=jax.ShapeDtypeStruct((n_chunks * n_out_pad, CW), jnp.float32),
        mesh=_mesh(),
        compiler_params=_SC_PARAMS,
        scratch_types=scratch,
    )
    def k(*refs):
        tabs = refs[:n_chunks]
        if pipe:
            (src_h, dst_h, out_h, acc, idxs, idxd, rows, zbuf, sem,
             idxs2, idxd2, rows2, sem2) = refs[n_chunks:]
        else:
            src_h, dst_h, out_h, acc, idxs, idxd, rows, zbuf, sem = refs[n_chunks:]
        cid = lax.axis_index("c")
        sid = lax.axis_index("s")

        def zb(i, _):
            zbuf[i, :] = jnp.zeros((L,), jnp.float32)
            return 0
        lax.fori_loop(0, ZB, zb, 0)

        base = sid * e_pt
        for c in range(n_chunks):
            on = (cid == (c % NC))

            @pl.when(on)
            def _():
                _zero_stripe(acc, zbuf, sid * stripe, stripe, ZB)
            plsc.subcore_barrier()

            @pl.when(on)
            def _():
                if pipe:
                    def body(p, _):
                        off0 = base + (2 * p) * eb
                        off1 = off0 + eb
                        pltpu.sync_copy(src_h.at[pl.ds(off0, eb)], idxs)
                        dA = pltpu.async_copy(tabs[c].at[idxs], rows, sem)
                        pltpu.sync_copy(dst_h.at[pl.ds(off0, eb)], idxd)
                        pltpu.sync_copy(src_h.at[pl.ds(off1, eb)], idxs2)
                        dB = pltpu.async_copy(tabs[c].at[idxs2], rows2, sem2)
                        pltpu.sync_copy(dst_h.at[pl.ds(off1, eb)], idxd2)
                        dA.wait()
                        pltpu.sync_copy(rows, acc.at[idxd], add=True)
                        dB.wait()
                        pltpu.sync_copy(rows2, acc.at[idxd2], add=True)
                        return 0
                    lax.fori_loop(0, n_pairs, body, 0)
                    for t in range(tail):
                        off = base + (2 * n_pairs + t) * eb
                        pltpu.sync_copy(src_h.at[pl.ds(off, eb)], idxs)
                        pltpu.sync_copy(dst_h.at[pl.ds(off, eb)], idxd)
                        pltpu.async_copy(tabs[c].at[idxs], rows, sem).wait()
                        pltpu.sync_copy(rows, acc.at[idxd], add=True)
                else:
                    def body(b, _):
                        off = base + b * eb
                        pltpu.sync_copy(src_h.at[pl.ds(off, eb)], idxs)
                        pltpu.sync_copy(dst_h.at[pl.ds(off, eb)], idxd)
                        pltpu.async_copy(tabs[c].at[idxs], rows, sem).wait()
                        pltpu.sync_copy(rows, acc.at[idxd], add=True)
                        return 0
                    lax.fori_loop(0, nb, body, 0)
            plsc.subcore_barrier()

            @pl.when(on)
            def _():
                _dump_stripe(acc, rows, out_h, sid * stripe,
                             c * n_out_pad + sid * stripe, stripe, eb)
            plsc.subcore_barrier()

    return k(*tables, src, dst)


def _translate_edges(cluster_pad, src, dst, trash, eb):
    n = cluster_pad.shape[0]
    e = src.shape[0]
    e_pw = e // NW
    nb = e_pw // eb

    @functools.partial(
        pl.kernel,
        out_type=(jax.ShapeDtypeStruct((e,), jnp.int32),
                  jax.ShapeDtypeStruct((e,), jnp.int32)),
        mesh=_mesh(),
        compiler_params=_SC_PARAMS,
        scratch_types=[
            pltpu.VMEM((n,), jnp.int32),
            pltpu.VMEM((eb,), jnp.int32),
            pltpu.VMEM((eb,), jnp.int32),
            pltpu.VMEM((eb,), jnp.int32),
            pltpu.VMEM((eb,), jnp.int32),
        ],
    )
    def k(cl_h, src_h, dst_h, nsrc_h, mdst_h, cl, idxs, idxd, obs, obd):
        wid = _wid()
        pltpu.sync_copy(cl_h, cl)
        base = wid * e_pw
        trash_v = jnp.full((L,), trash, jnp.int32)

        def body(b, _):
            off = base + b * eb
            pltpu.sync_copy(src_h.at[pl.ds(off, eb)], idxs)
            pltpu.sync_copy(dst_h.at[pl.ds(off, eb)], idxd)

            def inner(j, _):
                s16 = idxs[pl.ds(j * L, L)]
                d16 = idxd[pl.ds(j * L, L)]
                cs = plsc.load_gather(cl, [s16])
                cd = plsc.load_gather(cl, [d16])
                obs[pl.ds(j * L, L)] = cs
                obd[pl.ds(j * L, L)] = jnp.where(cs == cd, trash_v, cd)
                return 0

            lax.fori_loop(0, eb // L, inner, 0)
            pltpu.sync_copy(obs, nsrc_h.at[pl.ds(off, eb)])
            pltpu.sync_copy(obd, mdst_h.at[pl.ds(off, eb)])
            return 0

        lax.fori_loop(0, nb, body, 0)

    return k(cluster_pad, src, dst)


def _edge_dot(s_pad, src, dst, eb):
    n = s_pad.shape[0]
    e = src.shape[0]
    e_pw = e // NW
    nb = e_pw // eb

    @functools.partial(
        pl.kernel,
        out_type=jax.ShapeDtypeStruct((NW * L,), jnp.float32),
        mesh=_mesh(),
        compiler_params=_SC_PARAMS,
        scratch_types=[
            pltpu.VMEM((n,), jnp.float32),
            pltpu.VMEM((eb,), jnp.int32),
            pltpu.VMEM((eb,), jnp.int32),
            pltpu.VMEM((L,), jnp.float32),
        ],
    )
    def k(s_h, src_h, dst_h, out_h, sv, idxs, idxd, accb):
        wid = _wid()
        pltpu.sync_copy(s_h, sv)
        base = wid * e_pw

        def body(b, acc):
            off = base + b * eb
            pltpu.sync_copy(src_h.at[pl.ds(off, eb)], idxs)
            pltpu.sync_copy(dst_h.at[pl.ds(off, eb)], idxd)

            def inner(j, a):
                s16 = idxs[pl.ds(j * L, L)]
                d16 = idxd[pl.ds(j * L, L)]
                return a + (plsc.load_gather(sv, [s16])
                            * plsc.load_gather(sv, [d16]))

            return lax.fori_loop(0, eb // L, inner, acc)

        acc = lax.fori_loop(0, nb, body, jnp.zeros((L,), jnp.float32))
        accb[...] = acc
        pltpu.sync_copy(accb, out_h.at[pl.ds(wid * L, L)])

    return k(s_pad, src, dst)


def _rmw_max(acc, d16, v16, active):
    def cond(p):
        return jnp.any(p)

    def body(p):
        cur = plsc.load_gather(acc, [d16], mask=p)
        upd = p & (v16 > cur)
        plsc.store_scatter(acc, [d16], v16, mask=upd)
        chk = plsc.load_gather(acc, [d16], mask=upd)
        return upd & (chk < v16)

    lax.while_loop(cond, body, active)


def _hop_best(skey_pad, src, dst, eb):
    e = src.shape[0]
    e_pw = e // NW
    nb = e_pw // eb

    @functools.partial(
        pl.kernel,
        out_type=jax.ShapeDtypeStruct((NW * N_PAD,), jnp.float32),
        mesh=_mesh(),
        compiler_params=_SC_PARAMS,
        scratch_types=[
            pltpu.VMEM((N_PAD,), jnp.float32),
            pltpu.VMEM((eb,), jnp.int32),
            pltpu.VMEM((eb,), jnp.int32),
            pltpu.VMEM((eb,), jnp.float32),
            pltpu.SemaphoreType.DMA,
        ],
    )
    def k(skey_h, src_h, dst_h, out_h, acc, idxs, idxd, kv, sem):
        wid = _wid()
        _fill_1d(acc, N_PAD, NEG_INF, jnp.float32)
        base = wid * e_pw
        ones = jnp.ones((L,), jnp.bool_)

        def body(b, _):
            off = base + b * eb
            pltpu.sync_copy(src_h.at[pl.ds(off, eb)], idxs)
            pltpu.async_copy(skey_h.at[idxs], kv, sem).wait()
            pltpu.sync_copy(dst_h.at[pl.ds(off, eb)], idxd)

            def inner(j, _):
                d16 = idxd[pl.ds(j * L, L)]
                v16 = kv[pl.ds(j * L, L)]
                _rmw_max(acc, d16, v16, ones)
                return 0

            lax.fori_loop(0, eb // L, inner, 0)
            return 0

        lax.fori_loop(0, nb, body, 0)
        pltpu.sync_copy(acc, out_h.at[pl.ds(wid * N_PAD, N_PAD)])

    return k(skey_pad, src, dst)


def _hop_win(skey_pad, cluster_pad, best_pad, src, dst, eb):
    e = src.shape[0]
    e_pw = e // NW
    nb = e_pw // eb

    @functools.partial(
        pl.kernel,
        out_type=jax.ShapeDtypeStruct((NW * N_PAD,), jnp.int32),
        mesh=_mesh(),
        compiler_params=_SC_PARAMS,
        scratch_types=[
            pltpu.VMEM((N_PAD,), jnp.int32),
            pltpu.VMEM((eb,), jnp.int32),
            pltpu.VMEM((eb,), jnp.int32),
            pltpu.VMEM((eb,), jnp.float32),
            pltpu.VMEM((eb,), jnp.int32),
            pltpu.VMEM((eb,), jnp.float32),
            pltpu.SemaphoreType.DMA,
        ],
    )
    def k(skey_h, cl_h, best_h, src_h, dst_h, out_h,
          acc, idxs, idxd, kv, cv, bv, sem):
        wid = _wid()
        _fill_1d(acc, N_PAD, I32_MIN, jnp.int32)
        base = wid * e_pw
        ones = jnp.ones((L,), jnp.bool_)
        min_v = jnp.full((L,), I32_MIN, jnp.int32)

        def body(b, _):
            off = base + b * eb
            pltpu.sync_copy(src_h.at[pl.ds(off, eb)], idxs)
            pltpu.async_copy(skey_h.at[idxs], kv, sem).wait()
            pltpu.async_copy(cl_h.at[idxs], cv, sem).wait()
            pltpu.sync_copy(dst_h.at[pl.ds(off, eb)], idxd)
            pltpu.async_copy(best_h.at[idxd], bv, sem).wait()

            def inner(j, _):
                d16 = idxd[pl.ds(j * L, L)]
                k16 = kv[pl.ds(j * L, L)]
                c16 = cv[pl.ds(j * L, L)]
                b16 = bv[pl.ds(j * L, L)]
                v16 = jnp.where(k16 >= b16, c16, min_v)
                _rmw_max(acc, d16, v16, ones)
                return 0

            lax.fori_loop(0, eb // L, inner, 0)
            return 0

        lax.fori_loop(0, nb, body, 0)
        pltpu.sync_copy(acc, out_h.at[pl.ds(wid * N_PAD, N_PAD)])

    return k(skey_pad, cluster_pad, best_pad, src, dst)


NB = 2000


def _gin_body(x_ref, a0_ref, a1_ref, W1_ref, b1_ref, W2_ref, b2_ref, *out_refs):
    h0 = x_ref[...] + a0_ref[...] + a1_ref[...]
    h1 = jnp.maximum(jnp.dot(h0, W1_ref[...], preferred_element_type=jnp.float32)
                     + b1_ref[...], 0.0)
    h = jnp.dot(h1, W2_ref[...], preferred_element_type=jnp.float32) + b2_ref[...]
    for c, r in enumerate(out_refs):
        r[...] = h[:, c * CW:(c + 1) * CW]


def _gin_mlp_chunked(x, a0, a1, W1, b1, W2, b2, n_rows):
    grid = (n_rows // NB,)
    f_in = x.shape[1]
    f_mid = W1.shape[1]
    n_out_chunks = HID // CW
    return pl.pallas_call(
        _gin_body,
        grid=grid,
        in_specs=[
            pl.BlockSpec((NB, f_in), lambda i: (i, 0)),
            pl.BlockSpec((NB, f_in), lambda i: (i, 0)),
            pl.BlockSpec((NB, f_in), lambda i: (i, 0)),
            pl.BlockSpec((f_in, f_mid), lambda i: (0, 0)),
            pl.BlockSpec((1, f_mid), lambda i: (0, 0)),
            pl.BlockSpec((f_mid, HID), lambda i: (0, 0)),
            pl.BlockSpec((1, HID), lambda i: (0, 0)),
        ],
        out_specs=[pl.BlockSpec((NB, CW), lambda i: (i, 0))
                   for _ in range(n_out_chunks)],
        out_shape=[jax.ShapeDtypeStruct((n_rows, CW), jnp.float32)
                   for _ in range(n_out_chunks)],
    )(x, a0, a1, W1, b1.reshape(1, -1), W2, b2.reshape(1, -1))


def _score_body(h0_ref, h1_ref, h2_ref, h3_ref, g0_ref, g1_ref, g2_ref, g3_ref,
                deg_ref, Wmp_ref, bmp_ref, Wm1_ref, bm1_ref, Wm2_ref, bm2_ref,
                Ws_ref, bs_ref, s_ref, *hs_refs):
    h = jnp.concatenate([h0_ref[...], h1_ref[...], h2_ref[...], h3_ref[...]],
                        axis=1)
    agg = jnp.concatenate([g0_ref[...], g1_ref[...], g2_ref[...], g3_ref[...]],
                          axis=1)
    aggn = agg / jnp.maximum(deg_ref[...], 1.0)
    z = jnp.maximum(jnp.dot(h + aggn, Wmp_ref[...],
                            preferred_element_type=jnp.float32) + bmp_ref[...], 0.0)
    z = jnp.maximum(jnp.dot(z, Wm1_ref[...],
                            preferred_element_type=jnp.float32) + bm1_ref[...], 0.0)
    z = jnp.maximum(jnp.dot(z, Wm2_ref[...],
                            preferred_element_type=jnp.float32) + bm2_ref[...], 0.0)
    s = jnp.tanh(jnp.dot(z, Ws_ref[...],
                         preferred_element_type=jnp.float32) + bs_ref[...])
    s_ref[...] = s
    hs = h * s
    for c, r in enumerate(hs_refs):
        r[...] = hs[:, c * CW:(c + 1) * CW]


def _score_net(hc, aggc, deg, Wmp, bmp, Wm1, bm1, Wm2, bm2, Ws, bs):
    grid = (N_NODES // NB,)
    outs = pl.pallas_call(
        _score_body,
        grid=grid,
        in_specs=(
            [pl.BlockSpec((NB, CW), lambda i: (i, 0)) for _ in range(4)]
            + [pl.BlockSpec((NB, CW), lambda i: (i, 0)) for _ in range(4)]
            + [pl.BlockSpec((NB, 1), lambda i: (i, 0)),
               pl.BlockSpec((HID, HID), lambda i: (0, 0)),
               pl.BlockSpec((1, HID), lambda i: (0, 0)),
               pl.BlockSpec((HID, 32), lambda i: (0, 0)),
               pl.BlockSpec((1, 32), lambda i: (0, 0)),
               pl.BlockSpec((32, 32), lambda i: (0, 0)),
               pl.BlockSpec((1, 32), lambda i: (0, 0)),
               pl.BlockSpec((32, 1), lambda i: (0, 0)),
               pl.BlockSpec((1, 1), lambda i: (0, 0))]
        ),
        out_specs=[pl.BlockSpec((NB, 1), lambda i: (i, 0))]
        + [pl.BlockSpec((NB, CW), lambda i: (i, 0)) for _ in range(4)],
        out_shape=[jax.ShapeDtypeStruct((N_NODES, 1), jnp.float32)]
        + [jax.ShapeDtypeStruct((N_NODES, CW), jnp.float32) for _ in range(4)],
    )(*hc, *aggc, deg.reshape(-1, 1), Wmp, bmp.reshape(1, -1), Wm1,
      bm1.reshape(1, -1), Wm2, bm2.reshape(1, -1), Ws, bs.reshape(1, -1))
    return outs[0][:, 0], outs[1:]


def _logits_body(p0, p1, p2, p3, cnt_ref, Wl_ref, bl_ref, out_ref):
    pooled = jnp.concatenate([p0[...], p1[...], p2[...], p3[...]], axis=1)
    pooled = pooled / jnp.maximum(cnt_ref[...], 1.0)
    logits = jnp.dot(pooled, Wl_ref[...],
                     preferred_element_type=jnp.float32) + bl_ref[...]
    m = jnp.max(logits, axis=1, keepdims=True)
    sh = logits - m
    lse = jnp.log(jnp.sum(jnp.exp(sh), axis=1, keepdims=True))
    out_ref[...] = sh - lse


def _logits(pooled_chunks, cnt, Wl, bl):
    return pl.pallas_call(
        _logits_body,
        grid=(1,),
        in_specs=[pl.BlockSpec((G_PAD, CW), lambda i: (0, 0)) for _ in range(4)]
        + [pl.BlockSpec((G_PAD, 1), lambda i: (0, 0)),
           pl.BlockSpec((HID, N_CLASSES), lambda i: (0, 0)),
           pl.BlockSpec((1, N_CLASSES), lambda i: (0, 0))],
        out_specs=pl.BlockSpec((G_PAD, N_CLASSES), lambda i: (0, 0)),
        out_shape=jax.ShapeDtypeStruct((G_PAD, N_CLASSES), jnp.float32),
    )(*pooled_chunks, cnt, Wl, bl.reshape(1, -1))


def _pad_rows(a, n_pad):
    return jnp.pad(a, ((0, n_pad - a.shape[0]), (0, 0)))


def _pad_1d(a, n_pad, val=0):
    return jnp.pad(a, (0, n_pad - a.shape[0]), constant_values=val)


def kernel(x, edge_index, batch, W1, b1, W2, b2, Wmp, bmp, Wm1, bm1, Wm2, bm2,
           Ws, bs, V1, c1, V2, c2, Wl, bl):
    src, dst = edge_index[0], edge_index[1]
    n = x.shape[0]
    e = src.shape[0]

    x16 = jnp.concatenate(
        [x, jnp.ones((n, 1), jnp.float32), jnp.zeros((n, CW - F_IN - 1), jnp.float32)],
        axis=1)
    x16p = _pad_rows(x16, N_PAD)
    agg1p = _seg_sum_partial(x16p, src, dst, N_PAD, 1000)
    agg1p = agg1p.reshape(NC, N_PAD, CW)
    a0 = agg1p[0, :n, :]
    a1 = agg1p[1, :n, :]
    deg = a0[:, F_IN] + a1[:, F_IN]
    W1p = jnp.pad(W1, ((0, CW - F_IN), (0, 0)))
    hc = _gin_mlp_chunked(x16, a0, a1, W1p, b1, W2, b2, n)

    hcp = [_pad_rows(h, N_PAD) for h in hc]
    aggh = _seg_sum_chunks(hcp, src, dst, N_PAD, 800).reshape(4, N_PAD, CW)
    aggc = [aggh[c, :n, :] for c in range(4)]
    s, hs_chunks = _score_net(hc, aggc, deg, Wmp, bmp, Wm1, bm1, Wm2, bm2, Ws, bs)

    sp = _pad_1d(s, N_PAD)
    mc_parts = _edge_dot(sp, src, dst, 2000)
    mc_loss = BETA * jnp.sum(mc_parts) / jnp.float32(e)

    u = lax.bitcast_convert_type(s, jnp.uint32)
    flip = jnp.where(u >> 31 == 1, jnp.uint32(0xFFFFFFFF), jnp.uint32(0x80000000))
    desc = ~(u ^ flip)
    _, order = lax.sort((desc, jnp.arange(n, dtype=jnp.int32)), num_keys=1)
    idx = order[:K_POOL]
    cluster = jnp.full((n,), -1, jnp.int32).at[idx].set(
        jnp.arange(K_POOL, dtype=jnp.int32))

    for _ in range(N_HOPS):
        skey = jnp.where(cluster >= 0, s, NEG_INF)
        skp = _pad_1d(skey, N_PAD, NEG_INF)
        clp = _pad_1d(cluster, N_PAD, -1)
        bp = _hop_best(skp, src, dst, 2000).reshape(NW, N_PAD)
        best = jnp.max(bp, axis=0)
        wp = _hop_win(skp, clp, best, src, dst, 2000).reshape(NW, N_PAD)
        win = jnp.max(wp, axis=0)[:n]
        cluster = jnp.where(cluster >= 0, cluster, jnp.maximum(win, -1))
    cluster = jnp.where(cluster >= 0, cluster, 0)

    E_XP = N_PAD
    hs_pad = [_pad_rows(c, N_PAD) for c in hs_chunks]
    lin_idx = jnp.arange(E_XP, dtype=jnp.int32)
    xp_dst = _pad_1d(cluster, E_XP, K_POOL)
    x_pool = _seg_sum_chunks(hs_pad, lin_idx, xp_dst, K_PAD, 3128)
    x_pool = x_pool.reshape(4, K_PAD, CW)
    batch_pool = batch[idx]

    clp = _pad_1d(cluster, N_PAD, 0)
    nsrc, mdst = _translate_edges(clp, src, dst, K_POOL, 2000)
    xp_chunks = [x_pool[c] for c in range(4)]
    agg2 = _seg_sum_chunks(xp_chunks, nsrc, mdst, K_PAD, 800)
    agg2 = agg2.reshape(4, K_PAD, CW)
    xpc = jnp.concatenate([x_pool[c, :K_POOL, :] for c in range(4)], axis=1)
    a2 = jnp.concatenate([agg2[c, :K_POOL, :] for c in range(4)], axis=1)
    zeros_k = jnp.zeros((K_POOL, HID), jnp.float32)
    h2c = _gin_mlp_chunked(xpc, a2, zeros_k, V1, c1, V2, c2, K_POOL)

    h2p = [_pad_rows(c, K_PAD) for c in h2c]
    ones_chunk = jnp.zeros((K_PAD, CW), jnp.float32).at[:K_POOL, 0].set(1.0)
    lin_k = jnp.arange(K_PAD, dtype=jnp.int32)
    ro_dst = _pad_1d(batch_pool, K_PAD, N_GRAPHS)
    ro = _seg_sum_chunks(h2p + [ones_chunk], lin_k, ro_dst, G_PAD, 3128)
    ro = ro.reshape(5, G_PAD, CW)
    pooled_chunks = [ro[c] for c in range(4)]
    cnt = ro[4, :, 0:1]
    out = _logits(pooled_chunks, cnt, Wl, bl)
    return out[:N_GRAPHS], mc_loss
